# Initial kernel scaffold; baseline (speedup 1.0000x reference)
import jax
import jax.numpy as jnp
from jax.experimental import pallas as pl


def kernel(Hs, Hw, HS, w2s, s2s, S2s, gw_Ws, gw_Wd, gw_as, gw_ad, gw_b, gs_W, gs_as, gs_ad, gs_b, gS_Ws, gS_Wd, gS_as, gS_ad, gS_b, f1_W, f1_b, f2_W, f2_b, ffn_W1, ffn_b1, ffn_W2, ffn_b2):
    raise NotImplementedError("write your pallas kernel here")


# speedup vs baseline: 6.7289x; 6.7289x over previous
import functools

import jax
import jax.numpy as jnp
from jax import lax
from jax.experimental import pallas as pl
from jax.experimental.pallas import tpu as pltpu
from jax.experimental.pallas import tpu_sc as plsc

H = 8
C = 128
D = 128
HID = H * C

NC = 2
NS = 16
NW = NC * NS
CH = 128

NDP = 10112
ROWS_PER_TILE = NDP // NS


def _round_up(x, m):
    return (x + m - 1) // m * m


def _proj_body(x_ref, w_ref, a_ref, h_ref, al_ref):
    x = x_ref[...]
    h = jnp.dot(x, w_ref[...], preferred_element_type=jnp.float32)
    h_ref[...] = h
    al_ref[...] = jnp.dot(h, a_ref[...], preferred_element_type=jnp.float32)


def _proj(x, w, a2t, bm):
    n = x.shape[0]
    grid = (n // bm,)
    return pl.pallas_call(
        _proj_body,
        grid=grid,
        in_specs=[
            pl.BlockSpec((bm, D), lambda i: (i, 0)),
            pl.BlockSpec((D, HID), lambda i: (0, 0)),
            pl.BlockSpec((HID, 16), lambda i: (0, 0)),
        ],
        out_specs=[
            pl.BlockSpec((bm, HID), lambda i: (i, 0)),
            pl.BlockSpec((bm, 16), lambda i: (i, 0)),
        ],
        out_shape=[
            jax.ShapeDtypeStruct((n, HID), jnp.float32),
            jax.ShapeDtypeStruct((n, 16), jnp.float32),
        ],
    )(x, w, a2t)


def _mm16_body(x_ref, w_ref, o_ref):
    o_ref[...] = jnp.dot(x_ref[...], w_ref[...],
                         preferred_element_type=jnp.float32)


def _mm16(x, w, bm):
    n = x.shape[0]
    return pl.pallas_call(
        _mm16_body,
        grid=(n // bm,),
        in_specs=[
            pl.BlockSpec((bm, D), lambda i: (i, 0)),
            pl.BlockSpec((D, 16), lambda i: (0, 0)),
        ],
        out_specs=pl.BlockSpec((bm, 16), lambda i: (i, 0)),
        out_shape=jax.ShapeDtypeStruct((n, 16), jnp.float32),
    )(x, w)


def _gat_out(p_ref, den_ref, b):
    p = p_ref[0] + p_ref[1]
    den = (den_ref[0] + den_ref[1])[:, :8]
    bm = p.shape[0]
    den = den.reshape(bm, 8, 1) + 1e-16
    u = p / den
    u = u.reshape(bm, HID) + b
    return jnp.where(u > 0, u, jnp.exp(jnp.minimum(u, 0.0)) - 1.0)


def _post_body(hs_ref, pw_ref, dw_ref, ps_ref, ds_ref, pS_ref, dS_ref,
               bw_ref, bs_ref, bS_ref,
               f1a_ref, f1b_ref, f1bias_ref, f2a_ref, f2b_ref, f2bias_ref,
               w1_ref, b1_ref, w2_ref, b2_ref, o_ref):
    uw = _gat_out(pw_ref, dw_ref, bw_ref[...])
    us = _gat_out(ps_ref, ds_ref, bs_ref[...])
    uS = _gat_out(pS_ref, dS_ref, bS_ref[...])
    z1 = jax.nn.sigmoid(
        jnp.dot(uw, f1a_ref[...], preferred_element_type=jnp.float32)
        + jnp.dot(us, f1b_ref[...], preferred_element_type=jnp.float32)
        + f1bias_ref[...])
    u1 = z1 * uw + (1.0 - z1) * us
    z2 = jax.nn.sigmoid(
        jnp.dot(u1, f2a_ref[...], preferred_element_type=jnp.float32)
        + jnp.dot(uS, f2b_ref[...], preferred_element_type=jnp.float32)
        + f2bias_ref[...])
    u2 = z2 * u1 + (1.0 - z2) * uS
    hmid = jnp.maximum(
        jnp.dot(u2, w1_ref[...], preferred_element_type=jnp.float32)
        + b1_ref[...], 0.0)
    o_ref[...] = (hs_ref[...]
                  + jnp.dot(hmid, w2_ref[...],
                            preferred_element_type=jnp.float32)
                  + b2_ref[...])


def _post(hs, pw, dw, ps, ds, pS, dS, bw, bs, bS,
          f1a, f1b, f1bias, f2a, f2b, f2bias, w1, b1, w2, b2, bm):
    n = hs.shape[0]

    def pspec():
        return pl.BlockSpec((2, bm, 8, 128), lambda i: (0, i, 0, 0))

    def dspec():
        return pl.BlockSpec((2, bm, 16), lambda i: (0, i, 0))

    def full(shape):
        nd = len(shape)
        return pl.BlockSpec(shape, lambda i: (0,) * nd)

    return pl.pallas_call(
        _post_body,
        grid=(n // bm,),
        in_specs=[
            pl.BlockSpec((bm, D), lambda i: (i, 0)),
            pspec(), dspec(), pspec(), dspec(), pspec(), dspec(),
            full((HID,)), full((HID,)), full((HID,)),
            full((HID, HID)), full((HID, HID)), full((HID,)),
            full((HID, HID)), full((HID, HID)), full((HID,)),
            full((HID, D)), full((D,)), full((D, D)), full((D,)),
        ],
        out_specs=pl.BlockSpec((bm, D), lambda i: (i, 0)),
        out_shape=jax.ShapeDtypeStruct((n, D), jnp.float32),
    )(hs, pw, dw, ps, ds, pS, dS, bw, bs, bS,
      f1a, f1b, f1bias, f2a, f2b, f2bias, w1, b1, w2, b2)


def _lane_bcast(v, j):
    dn = lax.GatherDimensionNumbers(
        offset_dims=(), collapsed_slice_dims=(0,), start_index_map=(0,))
    idx = jnp.full((16, 1), j, jnp.int32)
    return lax.gather(v, idx, dn, (1,),
                      mode=lax.GatherScatterMode.PROMISE_IN_BOUNDS)

def _edge_kernel(ep, nsrc):
    epw = ep // NW
    nch = epw // CH
    mesh = plsc.VectorSubcoreMesh(core_axis_name="c", subcore_axis_name="s")

    def body(src_hbm, dst_hbm, als_hbm, ald_hbm, hsf_hbm, zeros_hbm,
             zeros16_hbm, e_hbm, den_hbm, p_hbm,
             src_v, dst_v, idx_v, als_b, ald_b, e_b, rows_b,
             den_sp, p_sp, sem, sem2):
        cid = lax.axis_index("c")
        sid = lax.axis_index("s")
        wid = sid * NC + cid
        r0 = sid * ROWS_PER_TILE

        pltpu.sync_copy(zeros16_hbm.at[pl.ds(r0, ROWS_PER_TILE)],
                        den_sp.at[pl.ds(r0, ROWS_PER_TILE)])
        plsc.subcore_barrier()

        def chunk1(kc, _):
            base = wid * epw + kc * CH
            pltpu.sync_copy(src_hbm.at[pl.ds(base, CH)], src_v)
            pltpu.sync_copy(dst_hbm.at[pl.ds(base, CH)], dst_v)
            cp1 = pltpu.async_copy(als_hbm.at[src_v], als_b, sem)
            cp2 = pltpu.async_copy(ald_hbm.at[dst_v], ald_b, sem2)
            cp1.wait()
            cp2.wait()

            def row(i, _):
                s = als_b[i] + ald_b[i]
                s = jnp.where(s > 0, s, 0.2 * s)
                e_b[i] = jnp.exp(s)
                return 0

            lax.fori_loop(0, CH, row, 0)
            pltpu.sync_copy(e_b, den_sp.at[dst_v], add=True)
            pltpu.sync_copy(e_b, e_hbm.at[pl.ds(base, CH)])
            return 0

        lax.fori_loop(0, nch, chunk1, 0)
        plsc.subcore_barrier()
        pltpu.sync_copy(den_sp.at[pl.ds(r0, ROWS_PER_TILE)],
                        den_hbm.at[cid, pl.ds(r0, ROWS_PER_TILE)])

        def head(h, _):
            pltpu.sync_copy(zeros_hbm.at[pl.ds(r0, ROWS_PER_TILE)],
                            p_sp.at[pl.ds(r0, ROWS_PER_TILE)])
            plsc.subcore_barrier()

            def chunk2(kc, _):
                base = wid * epw + kc * CH
                pltpu.sync_copy(src_hbm.at[pl.ds(base, CH)], src_v)
                pltpu.sync_copy(dst_hbm.at[pl.ds(base, CH)], dst_v)
                pltpu.sync_copy(e_hbm.at[pl.ds(base, CH)], e_b)

                def mkidx(j, _):
                    sl = pl.ds(j * 16, 16)
                    idx_v[sl] = src_v[sl] + h * nsrc
                    return 0

                lax.fori_loop(0, CH // 16, mkidx, 0)
                pltpu.async_copy(hsf_hbm.at[idx_v], rows_b, sem).wait()

                def row(i, _):
                    wv = _lane_bcast(e_b[i], h)
                    for j in range(8):
                        sl = pl.ds(j * 16, 16)
                        rows_b[i, sl] = rows_b[i, sl] * wv
                    return 0

                lax.fori_loop(0, CH, row, 0)
                pltpu.sync_copy(rows_b, p_sp.at[dst_v], add=True)
                return 0

            lax.fori_loop(0, nch, chunk2, 0)
            plsc.subcore_barrier()
            pltpu.sync_copy(p_sp.at[pl.ds(r0, ROWS_PER_TILE)],
                            p_hbm.at[cid, pl.ds(r0, ROWS_PER_TILE), h])
            plsc.subcore_barrier()
            return 0

        lax.fori_loop(0, H, head, 0)

    return pl.kernel(
        body,
        out_type=[
            jax.ShapeDtypeStruct((ep, 16), jnp.float32),
            jax.ShapeDtypeStruct((NC, NDP, 16), jnp.float32),
            jax.ShapeDtypeStruct((NC, NDP, 8, 128), jnp.float32),
        ],
        mesh=mesh,
        scratch_types=[
            pltpu.VMEM((CH,), jnp.int32),
            pltpu.VMEM((CH,), jnp.int32),
            pltpu.VMEM((CH,), jnp.int32),
            pltpu.VMEM((CH, 16), jnp.float32),
            pltpu.VMEM((CH, 16), jnp.float32),
            pltpu.VMEM((CH, 16), jnp.float32),
            pltpu.VMEM((CH, 128), jnp.float32),
            pltpu.VMEM_SHARED((NDP, 16), jnp.float32),
            pltpu.VMEM_SHARED((NDP, 128), jnp.float32),
            pltpu.SemaphoreType.DMA,
            pltpu.SemaphoreType.DMA,
        ],
        compiler_params=pltpu.CompilerParams(use_tc_tiling_on_sc=False),
    )


def _collapse(w, a):
    return jnp.einsum('dhc,hc->dh', w.reshape(D, H, C), a)


def _a2t(a_src, a_dst):
    z = jnp.zeros((H, C, 16), jnp.float32)
    z = z.at[jnp.arange(H), :, jnp.arange(H)].set(a_src)
    z = z.at[jnp.arange(H), :, 8 + jnp.arange(H)].set(a_dst)
    return z.reshape(HID, 16)


def _pad_edges(edge, ep, dummy_dst):
    e = edge.shape[1]
    src = jnp.pad(edge[0].astype(jnp.int32), (0, ep - e))
    dst = jnp.pad(edge[1].astype(jnp.int32), (0, ep - e),
                  constant_values=dummy_dst)
    return src, dst


def _headmajor(hlin):
    n = hlin.shape[0]
    return hlin.reshape(n, H, C).transpose(1, 0, 2).reshape(H * n, C)


def kernel(Hs, Hw, HS, w2s, s2s, S2s, gw_Ws, gw_Wd, gw_as, gw_ad, gw_b,
           gs_W, gs_as, gs_ad, gs_b, gS_Ws, gS_Wd, gS_as, gS_ad, gS_b,
           f1_W, f1_b, f2_W, f2_b, ffn_W1, ffn_b1, ffn_W2, ffn_b2):
    Ns = Hs.shape[0]
    NSec = HS.shape[0]
    zeros = jnp.zeros((NDP, 128), jnp.float32)
    zeros16 = jnp.zeros((NDP, 16), jnp.float32)

    hlw, alw = _proj(Hw[:Ns], gw_Ws, _a2t(gw_as, jnp.zeros_like(gw_as)), 1000)
    hls, als16 = _proj(Hs, gs_W, _a2t(gs_as, gs_ad), 1000)
    hlS, alS = _proj(HS, gS_Ws, _a2t(gS_as, jnp.zeros_like(gS_as)), 1000)

    wd16 = jnp.concatenate(
        [_collapse(gw_Wd, gw_ad), _collapse(gS_Wd, gS_ad)], axis=1)
    ald16 = _mm16(Hs, wd16, 1000)

    def pad_rows(x):
        return jnp.pad(x, ((0, NDP - x.shape[0]), (0, 0)))

    zpad = jnp.zeros((Ns, 8), jnp.float32)
    ald_w = pad_rows(jnp.concatenate([ald16[:, 0:8], zpad], axis=1))
    ald_s = pad_rows(jnp.concatenate([als16[:, 8:16], zpad], axis=1))
    ald_S = pad_rows(jnp.concatenate([ald16[:, 8:16], zpad], axis=1))

    def run_rel(edge, als_rows, ald_rows, hlin, nsrc):
        ep = _round_up(edge.shape[1], NW * CH)
        src, dst = _pad_edges(edge, ep, NDP - 1)
        _, den, p = _edge_kernel(ep, nsrc)(
            src, dst, als_rows, ald_rows, _headmajor(hlin), zeros, zeros16)
        return den, p

    den_w, p_w = run_rel(w2s, alw, ald_w, hlw, Ns)
    den_s, p_s = run_rel(s2s, als16, ald_s, hls, Ns)
    den_S, p_S = run_rel(S2s, alS, ald_S, hlS, NSec)

    return _post(Hs,
                 p_w[:, :Ns], den_w[:, :Ns], p_s[:, :Ns], den_s[:, :Ns],
                 p_S[:, :Ns], den_S[:, :Ns],
                 gw_b, gs_b, gS_b,
                 f1_W[:HID], f1_W[HID:], f1_b,
                 f2_W[:HID], f2_W[HID:], f2_b,
                 ffn_W1, ffn_b1, ffn_W2, ffn_b2, 400)


# speedup vs baseline: 7.1103x; 1.0567x over previous
import functools

import jax
import jax.numpy as jnp
from jax import lax
from jax.experimental import pallas as pl
from jax.experimental.pallas import tpu as pltpu
from jax.experimental.pallas import tpu_sc as plsc

H = 8
C = 128
D = 128
HID = H * C

NC = 2
NS = 16
NW = NC * NS
CH = 128

NDP = 10112
ROWS_PER_TILE = NDP // NS


def _round_up(x, m):
    return (x + m - 1) // m * m


def _proj_body(x_ref, w_ref, a_ref, h_ref, al_ref):
    x = x_ref[...]
    h = jnp.dot(x, w_ref[...], preferred_element_type=jnp.float32)
    h_ref[...] = h
    al_ref[...] = jnp.dot(h, a_ref[...], preferred_element_type=jnp.float32)


def _proj(x, w, a2t, bm):
    n = x.shape[0]
    grid = (n // bm,)
    return pl.pallas_call(
        _proj_body,
        grid=grid,
        in_specs=[
            pl.BlockSpec((bm, D), lambda i: (i, 0)),
            pl.BlockSpec((D, HID), lambda i: (0, 0)),
            pl.BlockSpec((HID, 16), lambda i: (0, 0)),
        ],
        out_specs=[
            pl.BlockSpec((bm, HID), lambda i: (i, 0)),
            pl.BlockSpec((bm, 16), lambda i: (i, 0)),
        ],
        out_shape=[
            jax.ShapeDtypeStruct((n, HID), jnp.float32),
            jax.ShapeDtypeStruct((n, 16), jnp.float32),
        ],
    )(x, w, a2t)


def _mm16_body(x_ref, w_ref, o_ref):
    o_ref[...] = jnp.dot(x_ref[...], w_ref[...],
                         preferred_element_type=jnp.float32)


def _mm16(x, w, bm):
    n = x.shape[0]
    return pl.pallas_call(
        _mm16_body,
        grid=(n // bm,),
        in_specs=[
            pl.BlockSpec((bm, D), lambda i: (i, 0)),
            pl.BlockSpec((D, 16), lambda i: (0, 0)),
        ],
        out_specs=pl.BlockSpec((bm, 16), lambda i: (i, 0)),
        out_shape=jax.ShapeDtypeStruct((n, 16), jnp.float32),
    )(x, w)


def _gat_out(p_ref, den_ref, b):
    p = p_ref[0] + p_ref[1]
    den = (den_ref[0] + den_ref[1])[:, :8]
    bm = p.shape[0]
    den = den.reshape(bm, 8, 1) + 1e-16
    u = p / den
    u = u.reshape(bm, HID) + b
    return jnp.where(u > 0, u, jnp.exp(jnp.minimum(u, 0.0)) - 1.0)


def _post_body(hs_ref, pw_ref, dw_ref, ps_ref, ds_ref, pS_ref, dS_ref,
               bw_ref, bs_ref, bS_ref,
               f1a_ref, f1b_ref, f1bias_ref, f2a_ref, f2b_ref, f2bias_ref,
               w1_ref, b1_ref, w2_ref, b2_ref, o_ref):
    uw = _gat_out(pw_ref, dw_ref, bw_ref[...])
    us = _gat_out(ps_ref, ds_ref, bs_ref[...])
    uS = _gat_out(pS_ref, dS_ref, bS_ref[...])
    z1 = jax.nn.sigmoid(
        jnp.dot(uw, f1a_ref[...], preferred_element_type=jnp.float32)
        + jnp.dot(us, f1b_ref[...], preferred_element_type=jnp.float32)
        + f1bias_ref[...])
    u1 = z1 * uw + (1.0 - z1) * us
    z2 = jax.nn.sigmoid(
        jnp.dot(u1, f2a_ref[...], preferred_element_type=jnp.float32)
        + jnp.dot(uS, f2b_ref[...], preferred_element_type=jnp.float32)
        + f2bias_ref[...])
    u2 = z2 * u1 + (1.0 - z2) * uS
    hmid = jnp.maximum(
        jnp.dot(u2, w1_ref[...], preferred_element_type=jnp.float32)
        + b1_ref[...], 0.0)
    o_ref[...] = (hs_ref[...]
                  + jnp.dot(hmid, w2_ref[...],
                            preferred_element_type=jnp.float32)
                  + b2_ref[...])


def _post(hs, pw, dw, ps, ds, pS, dS, bw, bs, bS,
          f1a, f1b, f1bias, f2a, f2b, f2bias, w1, b1, w2, b2, bm):
    n = hs.shape[0]

    def pspec():
        return pl.BlockSpec((2, bm, 8, 128), lambda i: (0, i, 0, 0))

    def dspec():
        return pl.BlockSpec((2, bm, 16), lambda i: (0, i, 0))

    def full(shape):
        nd = len(shape)
        return pl.BlockSpec(shape, lambda i: (0,) * nd)

    return pl.pallas_call(
        _post_body,
        grid=(n // bm,),
        in_specs=[
            pl.BlockSpec((bm, D), lambda i: (i, 0)),
            pspec(), dspec(), pspec(), dspec(), pspec(), dspec(),
            full((HID,)), full((HID,)), full((HID,)),
            full((HID, HID)), full((HID, HID)), full((HID,)),
            full((HID, HID)), full((HID, HID)), full((HID,)),
            full((HID, D)), full((D,)), full((D, D)), full((D,)),
        ],
        out_specs=pl.BlockSpec((bm, D), lambda i: (i, 0)),
        out_shape=jax.ShapeDtypeStruct((n, D), jnp.float32),
    )(hs, pw, dw, ps, ds, pS, dS, bw, bs, bS,
      f1a, f1b, f1bias, f2a, f2b, f2bias, w1, b1, w2, b2)


def _lane_bcast(v, j):
    dn = lax.GatherDimensionNumbers(
        offset_dims=(), collapsed_slice_dims=(0,), start_index_map=(0,))
    idx = jnp.full((16, 1), j, jnp.int32)
    return lax.gather(v, idx, dn, (1,),
                      mode=lax.GatherScatterMode.PROMISE_IN_BOUNDS)

def _logit_kernel(ep):
    epw = ep // NW
    nch = epw // CH
    mesh = plsc.VectorSubcoreMesh(core_axis_name="c", subcore_axis_name="s")

    def body(src_hbm, dst_hbm, als_hbm, ald_hbm, zeros16_hbm,
             e_hbm, den_hbm,
             srcm, dstm, als_b, ald_b, e_b, den_sp,
             semg0, semg1, seme0, seme1):
        cid = lax.axis_index("c")
        sid = lax.axis_index("s")
        wid = sid * NC + cid
        r0 = sid * ROWS_PER_TILE
        semg = (semg0, semg1)
        seme = (seme0, seme1)

        pltpu.async_copy(src_hbm.at[pl.ds(wid * nch, nch)], srcm, semg0)
        pltpu.async_copy(dst_hbm.at[pl.ds(wid * nch, nch)], dstm, semg1)
        pltpu.sync_copy(zeros16_hbm.at[pl.ds(r0, ROWS_PER_TILE)],
                        den_sp.at[pl.ds(r0, ROWS_PER_TILE)])
        pltpu.make_async_copy(src_hbm.at[pl.ds(wid * nch, nch)],
                              srcm, semg0).wait()
        pltpu.make_async_copy(dst_hbm.at[pl.ds(wid * nch, nch)],
                              dstm, semg1).wait()
        plsc.subcore_barrier()

        def gath1(k, b):
            pltpu.async_copy(als_hbm.at[srcm.at[k]], als_b.at[b], semg[b])
            pltpu.async_copy(ald_hbm.at[dstm.at[k]], ald_b.at[b], seme[b])

        def wait1(b):
            pltpu.make_async_copy(als_hbm.at[srcm.at[0]],
                                  als_b.at[b], semg[b]).wait()
            pltpu.make_async_copy(ald_hbm.at[dstm.at[0]],
                                  ald_b.at[b], seme[b]).wait()

        gath1(0, 0)

        def chunk1(g, _):
            for b in (0, 1):
                k = g * 2 + b

                @pl.when(k + 1 < nch)
                def _():
                    gath1(k + 1, 1 - b)

                wait1(b)
                ebuf = e_b.at[b]

                def row(i, _):
                    s = als_b[b, i] + ald_b[b, i]
                    s = jnp.where(s > 0, s, 0.2 * s)
                    ebuf[i] = jnp.exp(s)
                    return 0

                lax.fori_loop(0, CH, row, 0)
                pltpu.sync_copy(ebuf, den_sp.at[dstm.at[k]], add=True)
                pltpu.sync_copy(ebuf, e_hbm.at[pl.ds((wid * nch + k) * CH,
                                                     CH)])
            return 0

        lax.fori_loop(0, nch // 2, chunk1, 0)
        plsc.subcore_barrier()
        pltpu.sync_copy(den_sp.at[pl.ds(r0, ROWS_PER_TILE)],
                        den_hbm.at[cid, pl.ds(r0, ROWS_PER_TILE)])

    return pl.kernel(
        body,
        out_type=[
            jax.ShapeDtypeStruct((ep, 16), jnp.float32),
            jax.ShapeDtypeStruct((NC, NDP, 16), jnp.float32),
        ],
        mesh=mesh,
        scratch_types=[
            pltpu.VMEM((nch, CH), jnp.int32),
            pltpu.VMEM((nch, CH), jnp.int32),
            pltpu.VMEM((2, CH, 16), jnp.float32),
            pltpu.VMEM((2, CH, 16), jnp.float32),
            pltpu.VMEM((2, CH, 16), jnp.float32),
            pltpu.VMEM_SHARED((NDP, 16), jnp.float32),
        ] + [pltpu.SemaphoreType.DMA] * 4,
        compiler_params=pltpu.CompilerParams(use_tc_tiling_on_sc=False),
    )


def _msg_kernel(ep, nsrc):
    epw = ep // NW
    nch = epw // CH
    mesh = plsc.VectorSubcoreMesh(core_axis_name="c", subcore_axis_name="s")

    def body(src_hbm, dst_hbm, e_hbm, hsf_hbm, zeros_hbm, p_hbm,
             srcr, dstm, e_b, rows_b, p_sp,
             semg0, semg1, seme0, seme1, sems0, sems1, semr0, semr1):
        cid = lax.axis_index("c")
        sid = lax.axis_index("s")
        wid = sid * NC + cid
        r0 = sid * ROWS_PER_TILE
        semg = (semg0, semg1)
        seme = (seme0, seme1)
        sems = (sems0, sems1)
        semr = (semr0, semr1)

        pltpu.sync_copy(dst_hbm.at[pl.ds(wid * nch, nch)], dstm)

        def srcload(k, b, sem):
            pltpu.async_copy(src_hbm.at[pl.ds(wid * nch + k, 1)],
                             srcr.at[b], sem)

        def srcwait(b, sem):
            pltpu.make_async_copy(src_hbm.at[pl.ds(0, 1)],
                                  srcr.at[b], sem).wait()

        def head(h, _):
            pltpu.sync_copy(zeros_hbm.at[pl.ds(r0, ROWS_PER_TILE)],
                            p_sp.at[pl.ds(r0, ROWS_PER_TILE)])
            plsc.subcore_barrier()

            def mkoff(b):
                def mk16(j, _):
                    sl = pl.ds(j * 16, 16)
                    srcr[b, 0, sl] = srcr[b, 0, sl] + h * nsrc
                    return 0

                lax.fori_loop(0, CH // 16, mk16, 0)

            def gath2(k, b):
                pltpu.async_copy(hsf_hbm.at[srcr.at[b, 0]], rows_b.at[b],
                                 semg[b])
                pltpu.async_copy(
                    e_hbm.at[pl.ds((wid * nch + k) * CH, CH)],
                    e_b.at[b], seme[b])

            def wait2(b):
                pltpu.make_async_copy(hsf_hbm.at[srcr.at[0, 0]],
                                      rows_b.at[b], semg[b]).wait()
                pltpu.make_async_copy(
                    e_hbm.at[pl.ds(0, CH)], e_b.at[b], seme[b]).wait()

            def waitsc(b):
                pltpu.make_async_copy(rows_b.at[b],
                                      p_sp.at[dstm.at[0]], sems[b]).wait()

            srcload(0, 0, semr[0])
            srcload(1, 1, semr[1])
            srcwait(0, semr[0])
            mkoff(0)
            gath2(0, 0)

            def chunk2(g, _):
                for b in (0, 1):
                    k = g * 2 + b

                    @pl.when(k >= 1)
                    def _():
                        waitsc(1 - b)

                    @pl.when(k + 1 < nch)
                    def _():
                        srcwait(1 - b, semr[1 - b])
                        mkoff(1 - b)
                        gath2(k + 1, 1 - b)

                    wait2(b)
                    ebuf = e_b.at[b]
                    rbuf = rows_b.at[b]

                    def row(i, _):
                        wv = _lane_bcast(ebuf[i], h)
                        for j in range(8):
                            sl = pl.ds(j * 16, 16)
                            rbuf[i, sl] = rbuf[i, sl] * wv
                        return 0

                    lax.fori_loop(0, CH, row, 0)
                    pltpu.async_copy(rbuf, p_sp.at[dstm.at[k]], sems[b],
                                     add=True)

                    @pl.when(k + 2 < nch)
                    def _():
                        srcload(k + 2, b, semr[b])
                return 0

            lax.fori_loop(0, nch // 2, chunk2, 0)
            waitsc((nch - 1) % 2)
            plsc.subcore_barrier()
            pltpu.sync_copy(p_sp.at[pl.ds(r0, ROWS_PER_TILE)],
                            p_hbm.at[cid, pl.ds(r0, ROWS_PER_TILE), h])
            return 0

        lax.fori_loop(0, H, head, 0)

    return pl.kernel(
        body,
        out_type=jax.ShapeDtypeStruct((NC, NDP, 8, 128), jnp.float32),
        mesh=mesh,
        scratch_types=[
            pltpu.VMEM((2, 1, CH), jnp.int32),
            pltpu.VMEM((nch, CH), jnp.int32),
            pltpu.VMEM((2, CH, 16), jnp.float32),
            pltpu.VMEM((2, CH, 128), jnp.float32),
            pltpu.VMEM_SHARED((NDP, 128), jnp.float32),
        ] + [pltpu.SemaphoreType.DMA] * 8,
        compiler_params=pltpu.CompilerParams(use_tc_tiling_on_sc=False),
    )


def _collapse(w, a):
    return jnp.einsum('dhc,hc->dh', w.reshape(D, H, C), a)


def _a2t(a_src, a_dst):
    z = jnp.zeros((H, C, 16), jnp.float32)
    z = z.at[jnp.arange(H), :, jnp.arange(H)].set(a_src)
    z = z.at[jnp.arange(H), :, 8 + jnp.arange(H)].set(a_dst)
    return z.reshape(HID, 16)


def _pad_edges(edge, ep, dummy_dst):
    e = edge.shape[1]
    src = jnp.pad(edge[0].astype(jnp.int32), (0, ep - e))
    dst = jnp.pad(edge[1].astype(jnp.int32), (0, ep - e),
                  constant_values=dummy_dst)
    return src.reshape(ep // CH, CH), dst.reshape(ep // CH, CH)


def _headmajor(hlin):
    n = hlin.shape[0]
    return hlin.reshape(n, H, C).transpose(1, 0, 2).reshape(H * n, C)


def kernel(Hs, Hw, HS, w2s, s2s, S2s, gw_Ws, gw_Wd, gw_as, gw_ad, gw_b,
           gs_W, gs_as, gs_ad, gs_b, gS_Ws, gS_Wd, gS_as, gS_ad, gS_b,
           f1_W, f1_b, f2_W, f2_b, ffn_W1, ffn_b1, ffn_W2, ffn_b2):
    Ns = Hs.shape[0]
    NSec = HS.shape[0]
    zeros = jnp.zeros((NDP, 128), jnp.float32)
    zeros16 = jnp.zeros((NDP, 16), jnp.float32)

    hlw, alw = _proj(Hw[:Ns], gw_Ws, _a2t(gw_as, jnp.zeros_like(gw_as)), 1000)
    hls, als16 = _proj(Hs, gs_W, _a2t(gs_as, gs_ad), 1000)
    hlS, alS = _proj(HS, gS_Ws, _a2t(gS_as, jnp.zeros_like(gS_as)), 1000)

    wd16 = jnp.concatenate(
        [_collapse(gw_Wd, gw_ad), _collapse(gS_Wd, gS_ad)], axis=1)
    ald16 = _mm16(Hs, wd16, 1000)

    def pad_rows(x):
        return jnp.pad(x, ((0, NDP - x.shape[0]), (0, 0)))

    zpad = jnp.zeros((Ns, 8), jnp.float32)
    ald_w = pad_rows(jnp.concatenate([ald16[:, 0:8], zpad], axis=1))
    ald_s = pad_rows(jnp.concatenate([als16[:, 8:16], zpad], axis=1))
    ald_S = pad_rows(jnp.concatenate([ald16[:, 8:16], zpad], axis=1))

    def run_rel(edge, als_rows, ald_rows, hlin, nsrc):
        ep = _round_up(edge.shape[1], NW * CH * 2)
        src, dst = _pad_edges(edge, ep, NDP - 1)
        e, den = _logit_kernel(ep)(src, dst, als_rows, ald_rows, zeros16)
        p = _msg_kernel(ep, nsrc)(src, dst, e, _headmajor(hlin), zeros)
        return den, p

    den_w, p_w = run_rel(w2s, alw, ald_w, hlw, Ns)
    den_s, p_s = run_rel(s2s, als16, ald_s, hls, Ns)
    den_S, p_S = run_rel(S2s, alS, ald_S, hlS, NSec)

    return _post(Hs,
                 p_w[:, :Ns], den_w[:, :Ns], p_s[:, :Ns], den_s[:, :Ns],
                 p_S[:, :Ns], den_S[:, :Ns],
                 gw_b, gs_b, gS_b,
                 f1_W[:HID], f1_W[HID:], f1_b,
                 f2_W[:HID], f2_W[HID:], f2_b,
                 ffn_W1, ffn_b1, ffn_W2, ffn_b2, 400)


# speedup vs baseline: 7.5083x; 1.0560x over previous
import functools

import jax
import jax.numpy as jnp
from jax import lax
from jax.experimental import pallas as pl
from jax.experimental.pallas import tpu as pltpu
from jax.experimental.pallas import tpu_sc as plsc

H = 8
C = 128
D = 128
HID = H * C

NC = 2
NS = 16
NW = NC * NS
CH = 128

NDP = 10112
ROWS_PER_TILE = NDP // NS


def _round_up(x, m):
    return (x + m - 1) // m * m


def _proj_body(x_ref, w_ref, a_ref, h_ref, al_ref):
    x = x_ref[...]
    h = jnp.dot(x, w_ref[...], preferred_element_type=jnp.float32)
    h_ref[...] = h
    al_ref[...] = jnp.dot(h, a_ref[...], preferred_element_type=jnp.float32)


def _proj(x, w, a2t, bm):
    n = x.shape[0]
    grid = (n // bm,)
    return pl.pallas_call(
        _proj_body,
        grid=grid,
        in_specs=[
            pl.BlockSpec((bm, D), lambda i: (i, 0)),
            pl.BlockSpec((D, HID), lambda i: (0, 0)),
            pl.BlockSpec((HID, 16), lambda i: (0, 0)),
        ],
        out_specs=[
            pl.BlockSpec((bm, HID), lambda i: (i, 0)),
            pl.BlockSpec((bm, 16), lambda i: (i, 0)),
        ],
        out_shape=[
            jax.ShapeDtypeStruct((n, HID), jnp.float32),
            jax.ShapeDtypeStruct((n, 16), jnp.float32),
        ],
    )(x, w, a2t)


def _mm16_body(x_ref, w_ref, o_ref):
    o_ref[...] = jnp.dot(x_ref[...], w_ref[...],
                         preferred_element_type=jnp.float32)


def _mm16(x, w, bm):
    n = x.shape[0]
    return pl.pallas_call(
        _mm16_body,
        grid=(n // bm,),
        in_specs=[
            pl.BlockSpec((bm, D), lambda i: (i, 0)),
            pl.BlockSpec((D, 16), lambda i: (0, 0)),
        ],
        out_specs=pl.BlockSpec((bm, 16), lambda i: (i, 0)),
        out_shape=jax.ShapeDtypeStruct((n, 16), jnp.float32),
    )(x, w)


def _gat_out(p_ref, den_ref, b):
    p = p_ref[0] + p_ref[1]
    den = (den_ref[0] + den_ref[1])[:, :8]
    u = jnp.concatenate(
        [p[h] / (den[:, h:h + 1] + 1e-16) for h in range(8)], axis=-1)
    u = u + b
    return jnp.where(u > 0, u, jnp.exp(jnp.minimum(u, 0.0)) - 1.0)


def _post_body(hs_ref, pw_ref, dw_ref, ps_ref, ds_ref, pS_ref, dS_ref,
               bw_ref, bs_ref, bS_ref,
               f1a_ref, f1b_ref, f1bias_ref, f2a_ref, f2b_ref, f2bias_ref,
               w1_ref, b1_ref, w2_ref, b2_ref, o_ref):
    uw = _gat_out(pw_ref, dw_ref, bw_ref[...])
    us = _gat_out(ps_ref, ds_ref, bs_ref[...])
    uS = _gat_out(pS_ref, dS_ref, bS_ref[...])
    z1 = jax.nn.sigmoid(
        jnp.dot(uw, f1a_ref[...], preferred_element_type=jnp.float32)
        + jnp.dot(us, f1b_ref[...], preferred_element_type=jnp.float32)
        + f1bias_ref[...])
    u1 = z1 * uw + (1.0 - z1) * us
    z2 = jax.nn.sigmoid(
        jnp.dot(u1, f2a_ref[...], preferred_element_type=jnp.float32)
        + jnp.dot(uS, f2b_ref[...], preferred_element_type=jnp.float32)
        + f2bias_ref[...])
    u2 = z2 * u1 + (1.0 - z2) * uS
    hmid = jnp.maximum(
        jnp.dot(u2, w1_ref[...], preferred_element_type=jnp.float32)
        + b1_ref[...], 0.0)
    o_ref[...] = (hs_ref[...]
                  + jnp.dot(hmid, w2_ref[...],
                            preferred_element_type=jnp.float32)
                  + b2_ref[...])


def _post(hs, pw, dw, ps, ds, pS, dS, bw, bs, bS,
          f1a, f1b, f1bias, f2a, f2b, f2bias, w1, b1, w2, b2, bm):
    n = hs.shape[0]

    def pspec():
        return pl.BlockSpec((2, 8, bm, 128), lambda i: (0, 0, i, 0))

    def dspec():
        return pl.BlockSpec((2, bm, 16), lambda i: (0, i, 0))

    def full(shape):
        nd = len(shape)
        return pl.BlockSpec(shape, lambda i: (0,) * nd)

    return pl.pallas_call(
        _post_body,
        grid=(n // bm,),
        in_specs=[
            pl.BlockSpec((bm, D), lambda i: (i, 0)),
            pspec(), dspec(), pspec(), dspec(), pspec(), dspec(),
            full((HID,)), full((HID,)), full((HID,)),
            full((HID, HID)), full((HID, HID)), full((HID,)),
            full((HID, HID)), full((HID, HID)), full((HID,)),
            full((HID, D)), full((D,)), full((D, D)), full((D,)),
        ],
        out_specs=pl.BlockSpec((bm, D), lambda i: (i, 0)),
        out_shape=jax.ShapeDtypeStruct((n, D), jnp.float32),
    )(hs, pw, dw, ps, ds, pS, dS, bw, bs, bS,
      f1a, f1b, f1bias, f2a, f2b, f2bias, w1, b1, w2, b2)


def _lane_bcast(v, j):
    dn = lax.GatherDimensionNumbers(
        offset_dims=(), collapsed_slice_dims=(0,), start_index_map=(0,))
    idx = jnp.full((16, 1), j, jnp.int32)
    return lax.gather(v, idx, dn, (1,),
                      mode=lax.GatherScatterMode.PROMISE_IN_BOUNDS)

def _logit_kernel(ep, ndp):
    epw = ep // NW
    nch = epw // CH
    mesh = plsc.VectorSubcoreMesh(core_axis_name="c", subcore_axis_name="s")

    def body(src_hbm, dst_hbm, als_hbm, ald_hbm, zeros16_hbm,
             e_hbm, den_hbm,
             srcm, dstm, als_b, ald_b, e_b, den_sp,
             semg0, semg1, seme0, seme1):
        cid = lax.axis_index("c")
        sid = lax.axis_index("s")
        wid = sid * NC + cid
        rpt = ndp // NS
        r0 = sid * rpt
        semg = (semg0, semg1)
        seme = (seme0, seme1)

        pltpu.async_copy(src_hbm.at[pl.ds(wid * nch, nch)], srcm, semg0)
        pltpu.async_copy(dst_hbm.at[pl.ds(wid * nch, nch)], dstm, semg1)
        pltpu.sync_copy(zeros16_hbm.at[pl.ds(r0, rpt)],
                        den_sp.at[pl.ds(r0, rpt)])
        pltpu.make_async_copy(src_hbm.at[pl.ds(wid * nch, nch)],
                              srcm, semg0).wait()
        pltpu.make_async_copy(dst_hbm.at[pl.ds(wid * nch, nch)],
                              dstm, semg1).wait()
        plsc.subcore_barrier()

        def gath1(k, b):
            pltpu.async_copy(als_hbm.at[srcm.at[k]], als_b.at[b], semg[b])
            pltpu.async_copy(ald_hbm.at[dstm.at[k]], ald_b.at[b], seme[b])

        def wait1(b):
            pltpu.make_async_copy(als_hbm.at[srcm.at[0]],
                                  als_b.at[b], semg[b]).wait()
            pltpu.make_async_copy(ald_hbm.at[dstm.at[0]],
                                  ald_b.at[b], seme[b]).wait()

        gath1(0, 0)

        def chunk1(g, _):
            for b in (0, 1):
                k = g * 2 + b

                @pl.when(k + 1 < nch)
                def _():
                    gath1(k + 1, 1 - b)

                wait1(b)
                ebuf = e_b.at[b]

                def row(i, _):
                    s = als_b[b, i] + ald_b[b, i]
                    s = jnp.where(s > 0, s, 0.2 * s)
                    ebuf[i] = jnp.exp(s)
                    return 0

                lax.fori_loop(0, CH, row, 0)
                pltpu.sync_copy(ebuf, den_sp.at[dstm.at[k]], add=True)
                pltpu.sync_copy(ebuf, e_hbm.at[pl.ds((wid * nch + k) * CH,
                                                     CH)])
            return 0

        lax.fori_loop(0, nch // 2, chunk1, 0)
        plsc.subcore_barrier()
        pltpu.sync_copy(den_sp.at[pl.ds(r0, rpt)],
                        den_hbm.at[cid, pl.ds(r0, rpt)])

    return pl.kernel(
        body,
        out_type=[
            jax.ShapeDtypeStruct((ep, 16), jnp.float32),
            jax.ShapeDtypeStruct((NC, ndp, 16), jnp.float32),
        ],
        mesh=mesh,
        scratch_types=[
            pltpu.VMEM((nch, CH), jnp.int32),
            pltpu.VMEM((nch, CH), jnp.int32),
            pltpu.VMEM((2, CH, 16), jnp.float32),
            pltpu.VMEM((2, CH, 16), jnp.float32),
            pltpu.VMEM((2, CH, 16), jnp.float32),
            pltpu.VMEM_SHARED((ndp, 16), jnp.float32),
        ] + [pltpu.SemaphoreType.DMA] * 4,
        compiler_params=pltpu.CompilerParams(use_tc_tiling_on_sc=False),
    )


def _msg_kernel(ep, nsrc, ndp):
    epw = ep // NW
    nch = epw // CH
    mesh = plsc.VectorSubcoreMesh(core_axis_name="c", subcore_axis_name="s")

    def body(src_hbm, dst_hbm, e_hbm, hsf_hbm, zeros_hbm, p_hbm,
             srcr, dstm, e_b, rows_b, p_sp,
             semg0, semg1, seme0, seme1, sems0, sems1, semr0, semr1):
        cid = lax.axis_index("c")
        sid = lax.axis_index("s")
        wid = sid * NC + cid
        rpt = ndp // NS
        r0 = sid * rpt
        semg = (semg0, semg1)
        seme = (seme0, seme1)
        sems = (sems0, sems1)
        semr = (semr0, semr1)

        pltpu.sync_copy(dst_hbm.at[pl.ds(wid * nch, nch)], dstm)

        def srcload(k, b, sem):
            pltpu.async_copy(src_hbm.at[pl.ds(wid * nch + k, 1)],
                             srcr.at[b], sem)

        def srcwait(b, sem):
            pltpu.make_async_copy(src_hbm.at[pl.ds(0, 1)],
                                  srcr.at[b], sem).wait()

        def head(h, _):
            pltpu.sync_copy(zeros_hbm.at[pl.ds(r0, rpt)],
                            p_sp.at[pl.ds(r0, rpt)])
            plsc.subcore_barrier()

            def mkoff(b):
                def mk16(j, _):
                    sl = pl.ds(j * 16, 16)
                    srcr[b, 0, sl] = srcr[b, 0, sl] + h * nsrc
                    return 0

                lax.fori_loop(0, CH // 16, mk16, 0)

            def gath2(k, b):
                pltpu.async_copy(hsf_hbm.at[srcr.at[b, 0]], rows_b.at[b],
                                 semg[b])
                pltpu.async_copy(
                    e_hbm.at[pl.ds((wid * nch + k) * CH, CH)],
                    e_b.at[b], seme[b])

            def wait2(b):
                pltpu.make_async_copy(hsf_hbm.at[srcr.at[0, 0]],
                                      rows_b.at[b], semg[b]).wait()
                pltpu.make_async_copy(
                    e_hbm.at[pl.ds(0, CH)], e_b.at[b], seme[b]).wait()

            def waitsc(b):
                pltpu.make_async_copy(rows_b.at[b],
                                      p_sp.at[dstm.at[0]], sems[b]).wait()

            srcload(0, 0, semr[0])
            srcload(1, 1, semr[1])
            srcwait(0, semr[0])
            mkoff(0)
            gath2(0, 0)

            def chunk2(g, _):
                for b in (0, 1):
                    k = g * 2 + b

                    @pl.when(k >= 1)
                    def _():
                        waitsc(1 - b)

                    @pl.when(k + 1 < nch)
                    def _():
                        srcwait(1 - b, semr[1 - b])
                        mkoff(1 - b)
                        gath2(k + 1, 1 - b)

                    wait2(b)
                    ebuf = e_b.at[b]
                    rbuf = rows_b.at[b]

                    def row(i, _):
                        wv = _lane_bcast(ebuf[i], h)
                        for j in range(8):
                            sl = pl.ds(j * 16, 16)
                            rbuf[i, sl] = rbuf[i, sl] * wv
                        return 0

                    lax.fori_loop(0, CH, row, 0)
                    pltpu.async_copy(rbuf, p_sp.at[dstm.at[k]], sems[b],
                                     add=True)

                    @pl.when(k + 2 < nch)
                    def _():
                        srcload(k + 2, b, semr[b])
                return 0

            lax.fori_loop(0, nch // 2, chunk2, 0)
            waitsc((nch - 1) % 2)
            plsc.subcore_barrier()
            pltpu.sync_copy(p_sp.at[pl.ds(r0, rpt)],
                            p_hbm.at[cid, h, pl.ds(r0, rpt)])
            return 0

        lax.fori_loop(0, H, head, 0)

    return pl.kernel(
        body,
        out_type=jax.ShapeDtypeStruct((NC, 8, ndp, 128), jnp.float32),
        mesh=mesh,
        scratch_types=[
            pltpu.VMEM((2, 1, CH), jnp.int32),
            pltpu.VMEM((nch, CH), jnp.int32),
            pltpu.VMEM((2, CH, 16), jnp.float32),
            pltpu.VMEM((2, CH, 128), jnp.float32),
            pltpu.VMEM_SHARED((ndp, 128), jnp.float32),
        ] + [pltpu.SemaphoreType.DMA] * 8,
        compiler_params=pltpu.CompilerParams(use_tc_tiling_on_sc=False),
    )


def _collapse(w, a):
    return jnp.einsum('dhc,hc->dh', w.reshape(D, H, C), a)


def _a2t(a_src, a_dst):
    z = jnp.zeros((H, C, 16), jnp.float32)
    z = z.at[jnp.arange(H), :, jnp.arange(H)].set(a_src)
    z = z.at[jnp.arange(H), :, 8 + jnp.arange(H)].set(a_dst)
    return z.reshape(HID, 16)


def _pad_edges(edge, ep, dummy_dst):
    e = edge.shape[1]
    src = jnp.pad(edge[0].astype(jnp.int32), (0, ep - e))
    dst = jnp.pad(edge[1].astype(jnp.int32), (0, ep - e),
                  constant_values=dummy_dst)
    return src.reshape(ep // CH, CH), dst.reshape(ep // CH, CH)


def _headmajor(hlin):
    n = hlin.shape[0]
    return hlin.reshape(n, H, C).transpose(1, 0, 2).reshape(H * n, C)


def kernel(Hs, Hw, HS, w2s, s2s, S2s, gw_Ws, gw_Wd, gw_as, gw_ad, gw_b,
           gs_W, gs_as, gs_ad, gs_b, gS_Ws, gS_Wd, gS_as, gS_ad, gS_b,
           f1_W, f1_b, f2_W, f2_b, ffn_W1, ffn_b1, ffn_W2, ffn_b2):
    Ns = Hs.shape[0]
    NSec = HS.shape[0]
    zeros = jnp.zeros((NDP, 128), jnp.float32)
    zeros16 = jnp.zeros((NDP, 16), jnp.float32)

    hlw, alw = _proj(Hw[:Ns], gw_Ws, _a2t(gw_as, jnp.zeros_like(gw_as)), 1000)
    hls, als16 = _proj(Hs, gs_W, _a2t(gs_as, gs_ad), 1000)
    hlS, alS = _proj(HS, gS_Ws, _a2t(gS_as, jnp.zeros_like(gS_as)), 1000)

    wd16 = jnp.concatenate(
        [_collapse(gw_Wd, gw_ad), _collapse(gS_Wd, gS_ad)], axis=1)
    ald16 = _mm16(Hs, wd16, 1000)

    def pad_rows(x):
        return jnp.pad(x, ((0, NDP - x.shape[0]), (0, 0)))

    zpad = jnp.zeros((Ns, 8), jnp.float32)
    ald_w = pad_rows(jnp.concatenate([ald16[:, 0:8], zpad], axis=1))
    ald_s = pad_rows(jnp.concatenate([als16[:, 8:16], zpad], axis=1))
    ald_S = pad_rows(jnp.concatenate([ald16[:, 8:16], zpad], axis=1))

    def run_rel(edge, als_rows, ald_rows, hlin, nsrc, ndp):
        ep = _round_up(edge.shape[1], NW * CH * 2)
        src, dst = _pad_edges(edge, ep, ndp - 1)
        e, den = _logit_kernel(ep, ndp)(src, dst, als_rows, ald_rows,
                                        zeros16)
        p = _msg_kernel(ep, nsrc, ndp)(src, dst, e, _headmajor(hlin), zeros)
        return den, p

    den_w, p_w = run_rel(w2s, alw, ald_w, hlw, Ns, NDP)
    den_s, p_s = run_rel(s2s, als16, ald_s, hls, Ns, NDP)
    den_S, p_S = run_rel(S2s, alS, ald_S, hlS, NSec, 1024)
    p_S = jnp.pad(p_S[:, :, :NSec], ((0, 0), (0, 0), (0, Ns - NSec), (0, 0)))
    den_S = jnp.pad(den_S[:, :NSec], ((0, 0), (0, Ns - NSec), (0, 0)))

    return _post(Hs,
                 p_w[:, :, :Ns], den_w[:, :Ns], p_s[:, :, :Ns],
                 den_s[:, :Ns], p_S, den_S,
                 gw_b, gs_b, gS_b,
                 f1_W[:HID], f1_W[HID:], f1_b,
                 f2_W[:HID], f2_W[HID:], f2_b,
                 ffn_W1, ffn_b1, ffn_W2, ffn_b2, 400)


# speedup vs baseline: 7.5171x; 1.0012x over previous
import functools

import jax
import jax.numpy as jnp
from jax import lax
from jax.experimental import pallas as pl
from jax.experimental.pallas import tpu as pltpu
from jax.experimental.pallas import tpu_sc as plsc

H = 8
C = 128
D = 128
HID = H * C

NC = 2
NS = 16
NW = NC * NS
CH = 128

NDP = 10112
ROWS_PER_TILE = NDP // NS


def _round_up(x, m):
    return (x + m - 1) // m * m


def _proj_body(x_ref, w_ref, a_ref, h_ref, al_ref):
    x = x_ref[...]
    h = jnp.dot(x, w_ref[...], preferred_element_type=jnp.float32)
    h_ref[...] = h
    al_ref[...] = jnp.dot(h, a_ref[...], preferred_element_type=jnp.float32)


def _proj(x, w, a2t, bm):
    n = x.shape[0]
    grid = (n // bm,)
    return pl.pallas_call(
        _proj_body,
        grid=grid,
        in_specs=[
            pl.BlockSpec((bm, D), lambda i: (i, 0)),
            pl.BlockSpec((D, HID), lambda i: (0, 0)),
            pl.BlockSpec((HID, 16), lambda i: (0, 0)),
        ],
        out_specs=[
            pl.BlockSpec((bm, HID), lambda i: (i, 0)),
            pl.BlockSpec((bm, 16), lambda i: (i, 0)),
        ],
        out_shape=[
            jax.ShapeDtypeStruct((n, HID), jnp.float32),
            jax.ShapeDtypeStruct((n, 16), jnp.float32),
        ],
    )(x, w, a2t)


def _mm16_body(x_ref, w_ref, o_ref):
    o_ref[...] = jnp.dot(x_ref[...], w_ref[...],
                         preferred_element_type=jnp.float32)


def _mm16(x, w, bm):
    n = x.shape[0]
    return pl.pallas_call(
        _mm16_body,
        grid=(n // bm,),
        in_specs=[
            pl.BlockSpec((bm, D), lambda i: (i, 0)),
            pl.BlockSpec((D, 16), lambda i: (0, 0)),
        ],
        out_specs=pl.BlockSpec((bm, 16), lambda i: (i, 0)),
        out_shape=jax.ShapeDtypeStruct((n, 16), jnp.float32),
    )(x, w)


def _gat_out(p_ref, den_ref, b):
    p = p_ref[0] + p_ref[1]
    den = (den_ref[0] + den_ref[1])[:, :8]
    u = jnp.concatenate(
        [p[h] / (den[:, h:h + 1] + 1e-16) for h in range(8)], axis=-1)
    u = u + b
    return jnp.where(u > 0, u, jnp.exp(jnp.minimum(u, 0.0)) - 1.0)


def _post_body(hs_ref, pw_ref, dw_ref, ps_ref, ds_ref, pS_ref, dS_ref,
               bw_ref, bs_ref, bS_ref,
               f1a_ref, f1b_ref, f1bias_ref, f2a_ref, f2b_ref, f2bias_ref,
               w1_ref, b1_ref, w2_ref, b2_ref, o_ref):
    uw = _gat_out(pw_ref, dw_ref, bw_ref[...])
    us = _gat_out(ps_ref, ds_ref, bs_ref[...])
    uS = _gat_out(pS_ref, dS_ref, bS_ref[...])
    z1 = jax.nn.sigmoid(
        jnp.dot(uw, f1a_ref[...], preferred_element_type=jnp.float32)
        + jnp.dot(us, f1b_ref[...], preferred_element_type=jnp.float32)
        + f1bias_ref[...])
    u1 = z1 * uw + (1.0 - z1) * us
    z2 = jax.nn.sigmoid(
        jnp.dot(u1, f2a_ref[...], preferred_element_type=jnp.float32)
        + jnp.dot(uS, f2b_ref[...], preferred_element_type=jnp.float32)
        + f2bias_ref[...])
    u2 = z2 * u1 + (1.0 - z2) * uS
    hmid = jnp.maximum(
        jnp.dot(u2, w1_ref[...], preferred_element_type=jnp.float32)
        + b1_ref[...], 0.0)
    o_ref[...] = (hs_ref[...]
                  + jnp.dot(hmid, w2_ref[...],
                            preferred_element_type=jnp.float32)
                  + b2_ref[...])


def _post(hs, pw, dw, ps, ds, pS, dS, bw, bs, bS,
          f1a, f1b, f1bias, f2a, f2b, f2bias, w1, b1, w2, b2, bm):
    n = hs.shape[0]

    def pspec():
        return pl.BlockSpec((2, 8, bm, 128), lambda i: (0, 0, i, 0))

    def dspec():
        return pl.BlockSpec((2, bm, 16), lambda i: (0, i, 0))

    def full(shape):
        nd = len(shape)
        return pl.BlockSpec(shape, lambda i: (0,) * nd)

    return pl.pallas_call(
        _post_body,
        grid=(n // bm,),
        in_specs=[
            pl.BlockSpec((bm, D), lambda i: (i, 0)),
            pspec(), dspec(), pspec(), dspec(), pspec(), dspec(),
            full((HID,)), full((HID,)), full((HID,)),
            full((HID, HID)), full((HID, HID)), full((HID,)),
            full((HID, HID)), full((HID, HID)), full((HID,)),
            full((HID, D)), full((D,)), full((D, D)), full((D,)),
        ],
        out_specs=pl.BlockSpec((bm, D), lambda i: (i, 0)),
        out_shape=jax.ShapeDtypeStruct((n, D), jnp.float32),
    )(hs, pw, dw, ps, ds, pS, dS, bw, bs, bS,
      f1a, f1b, f1bias, f2a, f2b, f2bias, w1, b1, w2, b2)


def _lane_bcast(v, j):
    dn = lax.GatherDimensionNumbers(
        offset_dims=(), collapsed_slice_dims=(0,), start_index_map=(0,))
    idx = jnp.full((16, 1), j, jnp.int32)
    return lax.gather(v, idx, dn, (1,),
                      mode=lax.GatherScatterMode.PROMISE_IN_BOUNDS)

def _logit_kernel(ep, ndp):
    epw = ep // NW
    nch = epw // CH
    mesh = plsc.VectorSubcoreMesh(core_axis_name="c", subcore_axis_name="s")

    def body(src_hbm, dst_hbm, als_hbm, ald_hbm, zeros16_hbm,
             e_hbm, den_hbm,
             srcm, dstm, als_b, ald_b, e_b, den_sp,
             semg0, semg1, seme0, seme1):
        cid = lax.axis_index("c")
        sid = lax.axis_index("s")
        wid = sid * NC + cid
        rpt = ndp // NS
        r0 = sid * rpt
        semg = (semg0, semg1)
        seme = (seme0, seme1)

        pltpu.async_copy(src_hbm.at[pl.ds(wid * nch, nch)], srcm, semg0)
        pltpu.async_copy(dst_hbm.at[pl.ds(wid * nch, nch)], dstm, semg1)
        pltpu.sync_copy(zeros16_hbm.at[pl.ds(r0, rpt)],
                        den_sp.at[pl.ds(r0, rpt)])
        pltpu.make_async_copy(src_hbm.at[pl.ds(wid * nch, nch)],
                              srcm, semg0).wait()
        pltpu.make_async_copy(dst_hbm.at[pl.ds(wid * nch, nch)],
                              dstm, semg1).wait()
        plsc.subcore_barrier()

        def gath1(k, b):
            pltpu.async_copy(als_hbm.at[srcm.at[k]], als_b.at[b], semg[b])
            pltpu.async_copy(ald_hbm.at[dstm.at[k]], ald_b.at[b], seme[b])

        def wait1(b):
            pltpu.make_async_copy(als_hbm.at[srcm.at[0]],
                                  als_b.at[b], semg[b]).wait()
            pltpu.make_async_copy(ald_hbm.at[dstm.at[0]],
                                  ald_b.at[b], seme[b]).wait()

        gath1(0, 0)

        def chunk1(g, _):
            for b in (0, 1):
                k = g * 2 + b

                @pl.when(k + 1 < nch)
                def _():
                    gath1(k + 1, 1 - b)

                wait1(b)
                ebuf = e_b.at[b]

                def row(i2, _):
                    for u in range(2):
                        i = i2 * 2 + u
                        s = als_b[b, i] + ald_b[b, i]
                        s = jnp.where(s > 0, s, 0.2 * s)
                        ebuf[i] = jnp.exp(s)
                    return 0

                lax.fori_loop(0, CH // 2, row, 0)
                pltpu.sync_copy(ebuf, den_sp.at[dstm.at[k]], add=True)
                pltpu.sync_copy(ebuf, e_hbm.at[pl.ds((wid * nch + k) * CH,
                                                     CH)])
            return 0

        lax.fori_loop(0, nch // 2, chunk1, 0)
        plsc.subcore_barrier()
        pltpu.sync_copy(den_sp.at[pl.ds(r0, rpt)],
                        den_hbm.at[cid, pl.ds(r0, rpt)])

    return pl.kernel(
        body,
        out_type=[
            jax.ShapeDtypeStruct((ep, 16), jnp.float32),
            jax.ShapeDtypeStruct((NC, ndp, 16), jnp.float32),
        ],
        mesh=mesh,
        scratch_types=[
            pltpu.VMEM((nch, CH), jnp.int32),
            pltpu.VMEM((nch, CH), jnp.int32),
            pltpu.VMEM((2, CH, 16), jnp.float32),
            pltpu.VMEM((2, CH, 16), jnp.float32),
            pltpu.VMEM((2, CH, 16), jnp.float32),
            pltpu.VMEM_SHARED((ndp, 16), jnp.float32),
        ] + [pltpu.SemaphoreType.DMA] * 4,
        compiler_params=pltpu.CompilerParams(use_tc_tiling_on_sc=False),
    )


def _msg_kernel(ep, nsrc, ndp):
    epw = ep // NW
    nch = epw // CH
    mesh = plsc.VectorSubcoreMesh(core_axis_name="c", subcore_axis_name="s")

    def body(src_hbm, dst_hbm, e_hbm, hsf_hbm, zeros_hbm, p_hbm,
             srcr, dstm, e_b, rows_b, p_sp,
             semg0, semg1, seme0, seme1, sems0, sems1, semr0, semr1):
        cid = lax.axis_index("c")
        sid = lax.axis_index("s")
        wid = sid * NC + cid
        rpt = ndp // NS
        r0 = sid * rpt
        semg = (semg0, semg1)
        seme = (seme0, seme1)
        sems = (sems0, sems1)
        semr = (semr0, semr1)

        pltpu.sync_copy(dst_hbm.at[pl.ds(wid * nch, nch)], dstm)

        def srcload(k, b, sem):
            pltpu.async_copy(src_hbm.at[pl.ds(wid * nch + k, 1)],
                             srcr.at[b], sem)

        def srcwait(b, sem):
            pltpu.make_async_copy(src_hbm.at[pl.ds(0, 1)],
                                  srcr.at[b], sem).wait()

        def head(h, _):
            pltpu.sync_copy(zeros_hbm.at[pl.ds(r0, rpt)],
                            p_sp.at[pl.ds(r0, rpt)])
            plsc.subcore_barrier()

            def mkoff(b):
                def mk16(j, _):
                    sl = pl.ds(j * 16, 16)
                    srcr[b, 0, sl] = srcr[b, 0, sl] + h * nsrc
                    return 0

                lax.fori_loop(0, CH // 16, mk16, 0)

            def gath2(k, b):
                pltpu.async_copy(hsf_hbm.at[srcr.at[b, 0]], rows_b.at[b],
                                 semg[b])
                pltpu.async_copy(
                    e_hbm.at[pl.ds((wid * nch + k) * CH, CH)],
                    e_b.at[b], seme[b])

            def wait2(b):
                pltpu.make_async_copy(hsf_hbm.at[srcr.at[0, 0]],
                                      rows_b.at[b], semg[b]).wait()
                pltpu.make_async_copy(
                    e_hbm.at[pl.ds(0, CH)], e_b.at[b], seme[b]).wait()

            def waitsc(b):
                pltpu.make_async_copy(rows_b.at[b],
                                      p_sp.at[dstm.at[0]], sems[b]).wait()

            srcload(0, 0, semr[0])
            srcload(1, 1, semr[1])
            srcwait(0, semr[0])
            mkoff(0)
            gath2(0, 0)

            def chunk2(g, _):
                for b in (0, 1):
                    k = g * 2 + b

                    @pl.when(k >= 1)
                    def _():
                        waitsc(1 - b)

                    @pl.when(k + 1 < nch)
                    def _():
                        srcwait(1 - b, semr[1 - b])
                        mkoff(1 - b)
                        gath2(k + 1, 1 - b)

                    wait2(b)
                    ebuf = e_b.at[b]
                    rbuf = rows_b.at[b]

                    def row(i4, _):
                        for u in range(4):
                            i = i4 * 4 + u
                            wv = _lane_bcast(ebuf[i], h)
                            for j in range(8):
                                sl = pl.ds(j * 16, 16)
                                rbuf[i, sl] = rbuf[i, sl] * wv
                        return 0

                    lax.fori_loop(0, CH // 4, row, 0)
                    pltpu.async_copy(rbuf, p_sp.at[dstm.at[k]], sems[b],
                                     add=True)

                    @pl.when(k + 2 < nch)
                    def _():
                        srcload(k + 2, b, semr[b])
                return 0

            lax.fori_loop(0, nch // 2, chunk2, 0)
            waitsc((nch - 1) % 2)
            plsc.subcore_barrier()
            pltpu.sync_copy(p_sp.at[pl.ds(r0, rpt)],
                            p_hbm.at[cid, h, pl.ds(r0, rpt)])
            return 0

        lax.fori_loop(0, H, head, 0)

    return pl.kernel(
        body,
        out_type=jax.ShapeDtypeStruct((NC, 8, ndp, 128), jnp.float32),
        mesh=mesh,
        scratch_types=[
            pltpu.VMEM((2, 1, CH), jnp.int32),
            pltpu.VMEM((nch, CH), jnp.int32),
            pltpu.VMEM((2, CH, 16), jnp.float32),
            pltpu.VMEM((2, CH, 128), jnp.float32),
            pltpu.VMEM_SHARED((ndp, 128), jnp.float32),
        ] + [pltpu.SemaphoreType.DMA] * 8,
        compiler_params=pltpu.CompilerParams(use_tc_tiling_on_sc=False),
    )


def _collapse(w, a):
    return jnp.einsum('dhc,hc->dh', w.reshape(D, H, C), a)


def _a2t(a_src, a_dst):
    z = jnp.zeros((H, C, 16), jnp.float32)
    z = z.at[jnp.arange(H), :, jnp.arange(H)].set(a_src)
    z = z.at[jnp.arange(H), :, 8 + jnp.arange(H)].set(a_dst)
    return z.reshape(HID, 16)


def _pad_edges(edge, ep, dummy_dst):
    e = edge.shape[1]
    src = jnp.pad(edge[0].astype(jnp.int32), (0, ep - e))
    dst = jnp.pad(edge[1].astype(jnp.int32), (0, ep - e),
                  constant_values=dummy_dst)
    return src.reshape(ep // CH, CH), dst.reshape(ep // CH, CH)


def _headmajor(hlin):
    n = hlin.shape[0]
    return hlin.reshape(n, H, C).transpose(1, 0, 2).reshape(H * n, C)


def kernel(Hs, Hw, HS, w2s, s2s, S2s, gw_Ws, gw_Wd, gw_as, gw_ad, gw_b,
           gs_W, gs_as, gs_ad, gs_b, gS_Ws, gS_Wd, gS_as, gS_ad, gS_b,
           f1_W, f1_b, f2_W, f2_b, ffn_W1, ffn_b1, ffn_W2, ffn_b2):
    Ns = Hs.shape[0]
    NSec = HS.shape[0]
    zeros = jnp.zeros((NDP, 128), jnp.float32)
    zeros16 = jnp.zeros((NDP, 16), jnp.float32)

    hlw, alw = _proj(Hw[:Ns], gw_Ws, _a2t(gw_as, jnp.zeros_like(gw_as)), 1000)
    hls, als16 = _proj(Hs, gs_W, _a2t(gs_as, gs_ad), 1000)
    hlS, alS = _proj(HS, gS_Ws, _a2t(gS_as, jnp.zeros_like(gS_as)), 1000)

    wd16 = jnp.concatenate(
        [_collapse(gw_Wd, gw_ad), _collapse(gS_Wd, gS_ad)], axis=1)
    ald16 = _mm16(Hs, wd16, 1000)

    def pad_rows(x):
        return jnp.pad(x, ((0, NDP - x.shape[0]), (0, 0)))

    zpad = jnp.zeros((Ns, 8), jnp.float32)
    ald_w = pad_rows(jnp.concatenate([ald16[:, 0:8], zpad], axis=1))
    ald_s = pad_rows(jnp.concatenate([als16[:, 8:16], zpad], axis=1))
    ald_S = pad_rows(jnp.concatenate([ald16[:, 8:16], zpad], axis=1))

    def run_rel(edge, als_rows, ald_rows, hlin, nsrc, ndp):
        ep = _round_up(edge.shape[1], NW * CH * 2)
        src, dst = _pad_edges(edge, ep, ndp - 1)
        e, den = _logit_kernel(ep, ndp)(src, dst, als_rows, ald_rows,
                                        zeros16)
        p = _msg_kernel(ep, nsrc, ndp)(src, dst, e, _headmajor(hlin), zeros)
        return den, p

    den_w, p_w = run_rel(w2s, alw, ald_w, hlw, Ns, NDP)
    den_s, p_s = run_rel(s2s, als16, ald_s, hls, Ns, NDP)
    den_S, p_S = run_rel(S2s, alS, ald_S, hlS, NSec, 1024)
    p_S = jnp.pad(p_S[:, :, :NSec], ((0, 0), (0, 0), (0, Ns - NSec), (0, 0)))
    den_S = jnp.pad(den_S[:, :NSec], ((0, 0), (0, Ns - NSec), (0, 0)))

    return _post(Hs,
                 p_w[:, :, :Ns], den_w[:, :Ns], p_s[:, :, :Ns],
                 den_s[:, :Ns], p_S, den_S,
                 gw_b, gs_b, gS_b,
                 f1_W[:HID], f1_W[HID:], f1_b,
                 f2_W[:HID], f2_W[HID:], f2_b,
                 ffn_W1, ffn_b1, ffn_W2, ffn_b2, 400)


# speedup vs baseline: 10.2942x; 1.3694x over previous
import functools

import jax
import jax.numpy as jnp
from jax import lax
from jax.experimental import pallas as pl
from jax.experimental.pallas import tpu as pltpu
from jax.experimental.pallas import tpu_sc as plsc

H = 8
C = 128
D = 128
HID = H * C

NC = 2
NS = 16
NW = NC * NS
CH = 128
CH2 = 64

NDP = 10112
ROWS_PER_TILE = NDP // NS


def _round_up(x, m):
    return (x + m - 1) // m * m


def _proj_body(x_ref, w_ref, a_ref, h_ref, al_ref):
    x = x_ref[...]
    h = jnp.dot(x, w_ref[...], preferred_element_type=jnp.float32)
    h_ref[...] = h
    al_ref[...] = jnp.dot(h, a_ref[...], preferred_element_type=jnp.float32)


def _proj(x, w, a2t, bm):
    n = x.shape[0]
    grid = (n // bm,)
    return pl.pallas_call(
        _proj_body,
        grid=grid,
        in_specs=[
            pl.BlockSpec((bm, D), lambda i: (i, 0)),
            pl.BlockSpec((D, HID), lambda i: (0, 0)),
            pl.BlockSpec((HID, 16), lambda i: (0, 0)),
        ],
        out_specs=[
            pl.BlockSpec((bm, HID), lambda i: (i, 0)),
            pl.BlockSpec((bm, 16), lambda i: (i, 0)),
        ],
        out_shape=[
            jax.ShapeDtypeStruct((n, HID), jnp.float32),
            jax.ShapeDtypeStruct((n, 16), jnp.float32),
        ],
    )(x, w, a2t)


def _mm16_body(x_ref, w_ref, o_ref):
    o_ref[...] = jnp.dot(x_ref[...], w_ref[...],
                         preferred_element_type=jnp.float32)


def _mm16(x, w, bm):
    n = x.shape[0]
    return pl.pallas_call(
        _mm16_body,
        grid=(n // bm,),
        in_specs=[
            pl.BlockSpec((bm, D), lambda i: (i, 0)),
            pl.BlockSpec((D, 16), lambda i: (0, 0)),
        ],
        out_specs=pl.BlockSpec((bm, 16), lambda i: (i, 0)),
        out_shape=jax.ShapeDtypeStruct((n, 16), jnp.float32),
    )(x, w)


def _gat_out(p_ref, den_ref, b):
    p = p_ref[0] + p_ref[1]
    den = (den_ref[0] + den_ref[1])[:, :8]
    u = jnp.concatenate(
        [p[h] / (den[:, h:h + 1] + 1e-16) for h in range(8)], axis=-1)
    u = u + b
    return jnp.where(u > 0, u, jnp.exp(jnp.minimum(u, 0.0)) - 1.0)


def _post_body(hs_ref, pw_ref, dw_ref, ps_ref, ds_ref, pS_ref, dS_ref,
               bw_ref, bs_ref, bS_ref,
               f1a_ref, f1b_ref, f1bias_ref, f2a_ref, f2b_ref, f2bias_ref,
               w1_ref, b1_ref, w2_ref, b2_ref, o_ref):
    uw = _gat_out(pw_ref, dw_ref, bw_ref[...])
    us = _gat_out(ps_ref, ds_ref, bs_ref[...])
    uS = _gat_out(pS_ref, dS_ref, bS_ref[...])
    z1 = jax.nn.sigmoid(
        jnp.dot(uw, f1a_ref[...], preferred_element_type=jnp.float32)
        + jnp.dot(us, f1b_ref[...], preferred_element_type=jnp.float32)
        + f1bias_ref[...])
    u1 = z1 * uw + (1.0 - z1) * us
    z2 = jax.nn.sigmoid(
        jnp.dot(u1, f2a_ref[...], preferred_element_type=jnp.float32)
        + jnp.dot(uS, f2b_ref[...], preferred_element_type=jnp.float32)
        + f2bias_ref[...])
    u2 = z2 * u1 + (1.0 - z2) * uS
    hmid = jnp.maximum(
        jnp.dot(u2, w1_ref[...], preferred_element_type=jnp.float32)
        + b1_ref[...], 0.0)
    o_ref[...] = (hs_ref[...]
                  + jnp.dot(hmid, w2_ref[...],
                            preferred_element_type=jnp.float32)
                  + b2_ref[...])


def _post(hs, pw, dw, ps, ds, pS, dS, bw, bs, bS,
          f1a, f1b, f1bias, f2a, f2b, f2bias, w1, b1, w2, b2, bm):
    n = hs.shape[0]

    def pspec():
        return pl.BlockSpec((2, 8, bm, 128), lambda i: (0, 0, i, 0))

    def dspec():
        return pl.BlockSpec((2, bm, 16), lambda i: (0, i, 0))

    def full(shape):
        nd = len(shape)
        return pl.BlockSpec(shape, lambda i: (0,) * nd)

    return pl.pallas_call(
        _post_body,
        grid=(n // bm,),
        in_specs=[
            pl.BlockSpec((bm, D), lambda i: (i, 0)),
            pspec(), dspec(), pspec(), dspec(), pspec(), dspec(),
            full((HID,)), full((HID,)), full((HID,)),
            full((HID, HID)), full((HID, HID)), full((HID,)),
            full((HID, HID)), full((HID, HID)), full((HID,)),
            full((HID, D)), full((D,)), full((D, D)), full((D,)),
        ],
        out_specs=pl.BlockSpec((bm, D), lambda i: (i, 0)),
        out_shape=jax.ShapeDtypeStruct((n, D), jnp.float32),
    )(hs, pw, dw, ps, ds, pS, dS, bw, bs, bS,
      f1a, f1b, f1bias, f2a, f2b, f2bias, w1, b1, w2, b2)


def _lane_bcast(v, j):
    dn = lax.GatherDimensionNumbers(
        offset_dims=(), collapsed_slice_dims=(0,), start_index_map=(0,))
    idx = jnp.full((16, 1), j, jnp.int32)
    return lax.gather(v, idx, dn, (1,),
                      mode=lax.GatherScatterMode.PROMISE_IN_BOUNDS)

def _logit_kernel(ep, ndp):
    epw = ep // NW
    nch = epw // CH
    mesh = plsc.VectorSubcoreMesh(core_axis_name="c", subcore_axis_name="s")

    def body(src_hbm, dst_hbm, als_hbm, ald_hbm, zeros16_hbm,
             e_hbm, den_hbm,
             srcm, dstm, als_b, ald_b, e_b, den_sp,
             semg0, semg1, seme0, seme1):
        cid = lax.axis_index("c")
        sid = lax.axis_index("s")
        wid = sid * NC + cid
        rpt = ndp // NS
        r0 = sid * rpt
        semg = (semg0, semg1)
        seme = (seme0, seme1)

        pltpu.async_copy(src_hbm.at[pl.ds(wid * nch, nch)], srcm, semg0)
        pltpu.async_copy(dst_hbm.at[pl.ds(wid * nch, nch)], dstm, semg1)
        pltpu.sync_copy(zeros16_hbm.at[pl.ds(r0, rpt)],
                        den_sp.at[pl.ds(r0, rpt)])
        pltpu.make_async_copy(src_hbm.at[pl.ds(wid * nch, nch)],
                              srcm, semg0).wait()
        pltpu.make_async_copy(dst_hbm.at[pl.ds(wid * nch, nch)],
                              dstm, semg1).wait()
        plsc.subcore_barrier()

        def gath1(k, b):
            pltpu.async_copy(als_hbm.at[srcm.at[k]], als_b.at[b], semg[b])
            pltpu.async_copy(ald_hbm.at[dstm.at[k]], ald_b.at[b], seme[b])

        def wait1(b):
            pltpu.make_async_copy(als_hbm.at[srcm.at[0]],
                                  als_b.at[b], semg[b]).wait()
            pltpu.make_async_copy(ald_hbm.at[dstm.at[0]],
                                  ald_b.at[b], seme[b]).wait()

        gath1(0, 0)

        def chunk1(g, _):
            for b in (0, 1):
                k = g * 2 + b

                @pl.when(k + 1 < nch)
                def _():
                    gath1(k + 1, 1 - b)

                wait1(b)
                ebuf = e_b.at[b]

                def row(i2, _):
                    for u in range(2):
                        i = i2 * 2 + u
                        s = als_b[b, i] + ald_b[b, i]
                        s = jnp.where(s > 0, s, 0.2 * s)
                        ebuf[i] = jnp.exp(s)
                    return 0

                lax.fori_loop(0, CH // 2, row, 0)
                pltpu.sync_copy(ebuf, den_sp.at[dstm.at[k]], add=True)
                pltpu.sync_copy(ebuf, e_hbm.at[pl.ds((wid * nch + k) * CH,
                                                     CH)])
            return 0

        lax.fori_loop(0, nch // 2, chunk1, 0)
        plsc.subcore_barrier()
        pltpu.sync_copy(den_sp.at[pl.ds(r0, rpt)],
                        den_hbm.at[cid, pl.ds(r0, rpt)])

    return pl.kernel(
        body,
        out_type=[
            jax.ShapeDtypeStruct((ep, 16), jnp.float32),
            jax.ShapeDtypeStruct((NC, ndp, 16), jnp.float32),
        ],
        mesh=mesh,
        scratch_types=[
            pltpu.VMEM((nch, CH), jnp.int32),
            pltpu.VMEM((nch, CH), jnp.int32),
            pltpu.VMEM((2, CH, 16), jnp.float32),
            pltpu.VMEM((2, CH, 16), jnp.float32),
            pltpu.VMEM((2, CH, 16), jnp.float32),
            pltpu.VMEM_SHARED((ndp, 16), jnp.float32),
        ] + [pltpu.SemaphoreType.DMA] * 4,
        compiler_params=pltpu.CompilerParams(use_tc_tiling_on_sc=False),
    )


def _msg_kernel(ep, nsrc, ndp):
    epw = ep // NW
    nch = epw // CH2
    mesh = plsc.VectorSubcoreMesh(core_axis_name="c", subcore_axis_name="s")

    def body(src_hbm, dst_hbm, e_hbm, hsf_hbm, zeros_hbm, p_hbm,
             srcr, dstm, e_b, rows16, scaled, p_sp,
             semg0, semg1, seme0, seme1, sems0, sems1, semr0, semr1):
        cid = lax.axis_index("c")
        sid = lax.axis_index("s")
        wid = sid * NC + cid
        rpt = ndp // NS
        r0 = sid * rpt
        semg = (semg0, semg1)
        seme = (seme0, seme1)
        sems = (sems0, sems1)
        semr = (semr0, semr1)

        pltpu.sync_copy(dst_hbm.at[pl.ds(wid * nch, nch)], dstm)

        def srcload(k, b, sem):
            pltpu.async_copy(src_hbm.at[pl.ds(wid * nch + k, 1)],
                             srcr.at[b], sem)

        def srcwait(b, sem):
            pltpu.make_async_copy(src_hbm.at[pl.ds(0, 1)],
                                  srcr.at[b], sem).wait()

        def head(h, _):
            pltpu.sync_copy(zeros_hbm.at[pl.ds(r0, rpt)],
                            p_sp.at[pl.ds(r0, rpt)])
            plsc.subcore_barrier()

            def mkoff(b):
                def mk16(j, _):
                    sl = pl.ds(j * 16, 16)
                    srcr[b, 0, sl] = srcr[b, 0, sl] + h * nsrc
                    return 0

                lax.fori_loop(0, CH2 // 16, mk16, 0)

            def gath2(k, b):
                pltpu.async_copy(hsf_hbm.at[srcr.at[b, 0]], rows16.at[b],
                                 semg[b])
                pltpu.async_copy(
                    e_hbm.at[pl.ds((wid * nch + k) * CH2, CH2)],
                    e_b.at[b], seme[b])

            def wait2(b):
                pltpu.make_async_copy(hsf_hbm.at[srcr.at[0, 0]],
                                      rows16.at[b], semg[b]).wait()
                pltpu.make_async_copy(
                    e_hbm.at[pl.ds(0, CH2)], e_b.at[b], seme[b]).wait()

            def waitsc(b):
                pltpu.make_async_copy(scaled.at[b],
                                      p_sp.at[dstm.at[0]], sems[b]).wait()

            srcload(0, 0, semr[0])
            srcload(1, 1, semr[1])
            srcwait(0, semr[0])
            mkoff(0)
            gath2(0, 0)

            def chunk2(g, _):
                for b in (0, 1):
                    k = g * 2 + b

                    @pl.when(k >= 1)
                    def _():
                        waitsc(1 - b)

                    @pl.when(k + 1 < nch)
                    def _():
                        srcwait(1 - b, semr[1 - b])
                        mkoff(1 - b)
                        gath2(k + 1, 1 - b)

                    wait2(b)
                    ebuf = e_b.at[b]

                    def row(i2, _):
                        for u in range(2):
                            i = i2 * 2 + u
                            wv = _lane_bcast(ebuf[i], h)
                            for j in range(4):
                                x = rows16[b, i, pl.ds(j * 32, 32)]
                                lo, hi = plsc.unpack(
                                    x, format=plsc.PackFormat.INTERLEAVED)
                                scaled[b, i, pl.ds(j * 32, 16)] = lo * wv
                                scaled[b, i, pl.ds(j * 32 + 16, 16)] = (
                                    hi * wv)
                        return 0

                    lax.fori_loop(0, CH2 // 2, row, 0)
                    pltpu.async_copy(scaled.at[b], p_sp.at[dstm.at[k]],
                                     sems[b], add=True)

                    @pl.when(k + 2 < nch)
                    def _():
                        srcload(k + 2, b, semr[b])
                return 0

            lax.fori_loop(0, nch // 2, chunk2, 0)
            waitsc((nch - 1) % 2)
            plsc.subcore_barrier()
            pltpu.sync_copy(p_sp.at[pl.ds(r0, rpt)],
                            p_hbm.at[cid, h, pl.ds(r0, rpt)])
            return 0

        lax.fori_loop(0, H, head, 0)

    return pl.kernel(
        body,
        out_type=jax.ShapeDtypeStruct((NC, 8, ndp, 128), jnp.float32),
        mesh=mesh,
        scratch_types=[
            pltpu.VMEM((2, 1, CH2), jnp.int32),
            pltpu.VMEM((nch, CH2), jnp.int32),
            pltpu.VMEM((2, CH2, 16), jnp.float32),
            pltpu.VMEM((2, CH2, 128), jnp.bfloat16),
            pltpu.VMEM((2, CH2, 128), jnp.float32),
            pltpu.VMEM_SHARED((ndp, 128), jnp.float32),
        ] + [pltpu.SemaphoreType.DMA] * 8,
        compiler_params=pltpu.CompilerParams(use_tc_tiling_on_sc=False,
                                             needs_layout_passes=False),
    )


def _collapse(w, a):
    return jnp.einsum('dhc,hc->dh', w.reshape(D, H, C), a)


def _a2t(a_src, a_dst):
    z = jnp.zeros((H, C, 16), jnp.float32)
    z = z.at[jnp.arange(H), :, jnp.arange(H)].set(a_src)
    z = z.at[jnp.arange(H), :, 8 + jnp.arange(H)].set(a_dst)
    return z.reshape(HID, 16)


def _pad_edges(edge, ep, dummy_dst):
    e = edge.shape[1]
    src = jnp.pad(edge[0].astype(jnp.int32), (0, ep - e))
    dst = jnp.pad(edge[1].astype(jnp.int32), (0, ep - e),
                  constant_values=dummy_dst)
    return src, dst


def _headmajor16(hlin):
    n = hlin.shape[0]
    x = hlin.reshape(n, H, C).transpose(1, 0, 2).reshape(H * n, C)
    x = x.reshape(H * n, 4, 2, 16).transpose(0, 1, 3, 2)
    return x.reshape(H * n, C).astype(jnp.bfloat16)


def kernel(Hs, Hw, HS, w2s, s2s, S2s, gw_Ws, gw_Wd, gw_as, gw_ad, gw_b,
           gs_W, gs_as, gs_ad, gs_b, gS_Ws, gS_Wd, gS_as, gS_ad, gS_b,
           f1_W, f1_b, f2_W, f2_b, ffn_W1, ffn_b1, ffn_W2, ffn_b2):
    Ns = Hs.shape[0]
    NSec = HS.shape[0]
    zeros = jnp.zeros((NDP, 128), jnp.float32)
    zeros16 = jnp.zeros((NDP, 16), jnp.float32)

    hlw, alw = _proj(Hw[:Ns], gw_Ws, _a2t(gw_as, jnp.zeros_like(gw_as)), 1000)
    hls, als16 = _proj(Hs, gs_W, _a2t(gs_as, gs_ad), 1000)
    hlS, alS = _proj(HS, gS_Ws, _a2t(gS_as, jnp.zeros_like(gS_as)), 1000)

    wd16 = jnp.concatenate(
        [_collapse(gw_Wd, gw_ad), _collapse(gS_Wd, gS_ad)], axis=1)
    ald16 = _mm16(Hs, wd16, 1000)

    def pad_rows(x):
        return jnp.pad(x, ((0, NDP - x.shape[0]), (0, 0)))

    zpad = jnp.zeros((Ns, 8), jnp.float32)
    ald_w = pad_rows(jnp.concatenate([ald16[:, 0:8], zpad], axis=1))
    ald_s = pad_rows(jnp.concatenate([als16[:, 8:16], zpad], axis=1))
    ald_S = pad_rows(jnp.concatenate([ald16[:, 8:16], zpad], axis=1))

    def run_rel(edge, als_rows, ald_rows, hlin, nsrc, ndp):
        ep = _round_up(edge.shape[1], NW * CH * 2)
        src, dst = _pad_edges(edge, ep, ndp - 1)
        e, den = _logit_kernel(ep, ndp)(
            src.reshape(ep // CH, CH), dst.reshape(ep // CH, CH),
            als_rows, ald_rows, zeros16)
        p = _msg_kernel(ep, nsrc, ndp)(
            src.reshape(ep // CH2, CH2), dst.reshape(ep // CH2, CH2),
            e, _headmajor16(hlin), zeros)
        return den, p

    den_w, p_w = run_rel(w2s, alw, ald_w, hlw, Ns, NDP)
    den_s, p_s = run_rel(s2s, als16, ald_s, hls, Ns, NDP)
    den_S, p_S = run_rel(S2s, alS, ald_S, hlS, NSec, 1024)
    p_S = jnp.pad(p_S[:, :, :NSec], ((0, 0), (0, 0), (0, Ns - NSec), (0, 0)))
    den_S = jnp.pad(den_S[:, :NSec], ((0, 0), (0, Ns - NSec), (0, 0)))

    return _post(Hs,
                 p_w[:, :, :Ns], den_w[:, :Ns], p_s[:, :, :Ns],
                 den_s[:, :Ns], p_S, den_S,
                 gw_b, gs_b, gS_b,
                 f1_W[:HID], f1_W[HID:], f1_b,
                 f2_W[:HID], f2_W[HID:], f2_b,
                 ffn_W1, ffn_b1, ffn_W2, ffn_b2, 400)


# speedup vs baseline: 11.5593x; 1.1229x over previous
import functools

import jax
import jax.numpy as jnp
from jax import lax
from jax.experimental import pallas as pl
from jax.experimental.pallas import tpu as pltpu
from jax.experimental.pallas import tpu_sc as plsc

H = 8
C = 128
D = 128
HID = H * C

NC = 2
NS = 16
NW = NC * NS
CH = 128
CH2 = 64

NDP = 10112
ROWS_PER_TILE = NDP // NS


def _round_up(x, m):
    return (x + m - 1) // m * m


def _proj_body(x_ref, w_ref, a_ref, h_ref, al_ref):
    x = x_ref[...]
    h = jnp.dot(x, w_ref[...], preferred_element_type=jnp.float32)
    h_ref[...] = h
    al_ref[...] = jnp.dot(h, a_ref[...], preferred_element_type=jnp.float32)


def _proj(x, w, a2t, bm):
    n = x.shape[0]
    grid = (n // bm,)
    return pl.pallas_call(
        _proj_body,
        grid=grid,
        in_specs=[
            pl.BlockSpec((bm, D), lambda i: (i, 0)),
            pl.BlockSpec((D, HID), lambda i: (0, 0)),
            pl.BlockSpec((HID, 16), lambda i: (0, 0)),
        ],
        out_specs=[
            pl.BlockSpec((bm, HID), lambda i: (i, 0)),
            pl.BlockSpec((bm, 16), lambda i: (i, 0)),
        ],
        out_shape=[
            jax.ShapeDtypeStruct((n, HID), jnp.float32),
            jax.ShapeDtypeStruct((n, 16), jnp.float32),
        ],
    )(x, w, a2t)


def _mm16_body(x_ref, w_ref, o_ref):
    o_ref[...] = jnp.dot(x_ref[...], w_ref[...],
                         preferred_element_type=jnp.float32)


def _mm16(x, w, bm):
    n = x.shape[0]
    return pl.pallas_call(
        _mm16_body,
        grid=(n // bm,),
        in_specs=[
            pl.BlockSpec((bm, D), lambda i: (i, 0)),
            pl.BlockSpec((D, 16), lambda i: (0, 0)),
        ],
        out_specs=pl.BlockSpec((bm, 16), lambda i: (i, 0)),
        out_shape=jax.ShapeDtypeStruct((n, 16), jnp.float32),
    )(x, w)


def _gat_out(p_ref, den_ref, b):
    p = p_ref[0] + p_ref[1]
    den = (den_ref[0] + den_ref[1])[:, :8]
    u = jnp.concatenate(
        [p[h] / (den[:, h:h + 1] + 1e-16) for h in range(8)], axis=-1)
    u = u + b
    return jnp.where(u > 0, u, jnp.exp(jnp.minimum(u, 0.0)) - 1.0)


def _post_body(hs_ref, pw_ref, dw_ref, ps_ref, ds_ref, pS_ref, dS_ref,
               bw_ref, bs_ref, bS_ref,
               f1a_ref, f1b_ref, f1bias_ref, f2a_ref, f2b_ref, f2bias_ref,
               w1_ref, b1_ref, w2_ref, b2_ref, o_ref):
    uw = _gat_out(pw_ref, dw_ref, bw_ref[...])
    us = _gat_out(ps_ref, ds_ref, bs_ref[...])
    uS = _gat_out(pS_ref, dS_ref, bS_ref[...])
    z1 = jax.nn.sigmoid(
        jnp.dot(uw, f1a_ref[...], preferred_element_type=jnp.float32)
        + jnp.dot(us, f1b_ref[...], preferred_element_type=jnp.float32)
        + f1bias_ref[...])
    u1 = z1 * uw + (1.0 - z1) * us
    z2 = jax.nn.sigmoid(
        jnp.dot(u1, f2a_ref[...], preferred_element_type=jnp.float32)
        + jnp.dot(uS, f2b_ref[...], preferred_element_type=jnp.float32)
        + f2bias_ref[...])
    u2 = z2 * u1 + (1.0 - z2) * uS
    hmid = jnp.maximum(
        jnp.dot(u2, w1_ref[...], preferred_element_type=jnp.float32)
        + b1_ref[...], 0.0)
    o_ref[...] = (hs_ref[...]
                  + jnp.dot(hmid, w2_ref[...],
                            preferred_element_type=jnp.float32)
                  + b2_ref[...])


def _post(hs, pw, dw, ps, ds, pS, dS, bw, bs, bS,
          f1a, f1b, f1bias, f2a, f2b, f2bias, w1, b1, w2, b2, bm):
    n = hs.shape[0]

    def pspec():
        return pl.BlockSpec((2, 8, bm, 128), lambda i: (0, 0, i, 0))

    def dspec():
        return pl.BlockSpec((2, bm, 16), lambda i: (0, i, 0))

    def full(shape):
        nd = len(shape)
        return pl.BlockSpec(shape, lambda i: (0,) * nd)

    return pl.pallas_call(
        _post_body,
        grid=(n // bm,),
        in_specs=[
            pl.BlockSpec((bm, D), lambda i: (i, 0)),
            pspec(), dspec(), pspec(), dspec(), pspec(), dspec(),
            full((HID,)), full((HID,)), full((HID,)),
            full((HID, HID)), full((HID, HID)), full((HID,)),
            full((HID, HID)), full((HID, HID)), full((HID,)),
            full((HID, D)), full((D,)), full((D, D)), full((D,)),
        ],
        out_specs=pl.BlockSpec((bm, D), lambda i: (i, 0)),
        out_shape=jax.ShapeDtypeStruct((n, D), jnp.float32),
    )(hs, pw, dw, ps, ds, pS, dS, bw, bs, bS,
      f1a, f1b, f1bias, f2a, f2b, f2bias, w1, b1, w2, b2)


def _lane_bcast(v, j):
    dn = lax.GatherDimensionNumbers(
        offset_dims=(), collapsed_slice_dims=(0,), start_index_map=(0,))
    idx = jnp.full((16, 1), j, jnp.int32)
    return lax.gather(v, idx, dn, (1,),
                      mode=lax.GatherScatterMode.PROMISE_IN_BOUNDS)

def _logit_kernel(ep, ndp):
    epw = ep // NW
    nch = epw // CH
    mesh = plsc.VectorSubcoreMesh(core_axis_name="c", subcore_axis_name="s")

    def body(src_hbm, dst_hbm, als_hbm, ald_hbm, zeros16_hbm,
             e_hbm, den_hbm,
             srcm, dstm, als_b, ald_b, e_b, den_sp,
             semg0, semg1, seme0, seme1):
        cid = lax.axis_index("c")
        sid = lax.axis_index("s")
        wid = sid * NC + cid
        rpt = ndp // NS
        r0 = sid * rpt
        semg = (semg0, semg1)
        seme = (seme0, seme1)

        pltpu.async_copy(src_hbm.at[pl.ds(wid * nch, nch)], srcm, semg0)
        pltpu.async_copy(dst_hbm.at[pl.ds(wid * nch, nch)], dstm, semg1)
        pltpu.sync_copy(zeros16_hbm.at[pl.ds(r0, rpt)],
                        den_sp.at[pl.ds(r0, rpt)])
        pltpu.make_async_copy(src_hbm.at[pl.ds(wid * nch, nch)],
                              srcm, semg0).wait()
        pltpu.make_async_copy(dst_hbm.at[pl.ds(wid * nch, nch)],
                              dstm, semg1).wait()
        plsc.subcore_barrier()

        def gath1(k, b):
            pltpu.async_copy(als_hbm.at[srcm.at[k]], als_b.at[b], semg[b])
            pltpu.async_copy(ald_hbm.at[dstm.at[k]], ald_b.at[b], seme[b])

        def wait1(b):
            pltpu.make_async_copy(als_hbm.at[srcm.at[0]],
                                  als_b.at[b], semg[b]).wait()
            pltpu.make_async_copy(ald_hbm.at[dstm.at[0]],
                                  ald_b.at[b], seme[b]).wait()

        gath1(0, 0)

        def chunk1(g, _):
            for b in (0, 1):
                k = g * 2 + b

                @pl.when(k + 1 < nch)
                def _():
                    gath1(k + 1, 1 - b)

                wait1(b)
                ebuf = e_b.at[b]

                def row(i2, _):
                    for u in range(2):
                        i = i2 * 2 + u
                        s = als_b[b, i] + ald_b[b, i]
                        s = jnp.where(s > 0, s, 0.2 * s)
                        ebuf[i] = jnp.exp(s)
                    return 0

                lax.fori_loop(0, CH // 2, row, 0)
                pltpu.sync_copy(ebuf, den_sp.at[dstm.at[k]], add=True)
                pltpu.sync_copy(ebuf, e_hbm.at[pl.ds((wid * nch + k) * CH,
                                                     CH)])
            return 0

        lax.fori_loop(0, nch // 2, chunk1, 0)
        plsc.subcore_barrier()
        pltpu.sync_copy(den_sp.at[pl.ds(r0, rpt)],
                        den_hbm.at[cid, pl.ds(r0, rpt)])

    return pl.kernel(
        body,
        out_type=[
            jax.ShapeDtypeStruct((ep, 16), jnp.float32),
            jax.ShapeDtypeStruct((NC, ndp, 16), jnp.float32),
        ],
        mesh=mesh,
        scratch_types=[
            pltpu.VMEM((nch, CH), jnp.int32),
            pltpu.VMEM((nch, CH), jnp.int32),
            pltpu.VMEM((2, CH, 16), jnp.float32),
            pltpu.VMEM((2, CH, 16), jnp.float32),
            pltpu.VMEM((2, CH, 16), jnp.float32),
            pltpu.VMEM_SHARED((ndp, 16), jnp.float32),
        ] + [pltpu.SemaphoreType.DMA] * 4,
        compiler_params=pltpu.CompilerParams(use_tc_tiling_on_sc=False),
    )


def _msg_kernel(ep, nsrc, ndp):
    epw = ep // NW
    nch = epw // CH2
    mesh = plsc.VectorSubcoreMesh(core_axis_name="c", subcore_axis_name="s")

    def body(src_hbm, dst_hbm, e_hbm, hsf_hbm, zeros_hbm, p_hbm,
             srcr, dstm, e_b, rows16, scaled, p_sp,
             semg0, semg1, seme0, seme1, sems0, sems1, semr0, semr1):
        cid = lax.axis_index("c")
        sid = lax.axis_index("s")
        wid = sid * NC + cid
        rpt = ndp // NS
        r0 = sid * rpt
        semg = (semg0, semg1)
        seme = (seme0, seme1)
        sems = (sems0, sems1)
        semr = (semr0, semr1)

        pltpu.sync_copy(dst_hbm.at[pl.ds(wid * nch, nch)], dstm)

        def srcload(k, b, sem):
            pltpu.async_copy(src_hbm.at[pl.ds(wid * nch + k, 1)],
                             srcr.at[b], sem)

        def srcwait(b, sem):
            pltpu.make_async_copy(src_hbm.at[pl.ds(0, 1)],
                                  srcr.at[b], sem).wait()

        def head(h, _):
            pltpu.sync_copy(zeros_hbm.at[pl.ds(r0, rpt)],
                            p_sp.at[pl.ds(r0, rpt)])
            plsc.subcore_barrier()

            def mkoff(b):
                def mk16(j, _):
                    sl = pl.ds(j * 16, 16)
                    srcr[b, 0, sl] = srcr[b, 0, sl] + h * nsrc
                    return 0

                lax.fori_loop(0, CH2 // 16, mk16, 0)

            def gath2(k, b):
                pltpu.async_copy(hsf_hbm.at[srcr.at[b, 0]], rows16.at[b],
                                 semg[b])
                pltpu.async_copy(
                    e_hbm.at[pl.ds((wid * nch + k) * CH2, CH2)],
                    e_b.at[b], seme[b])

            def wait2(b):
                pltpu.make_async_copy(hsf_hbm.at[srcr.at[0, 0]],
                                      rows16.at[b], semg[b]).wait()
                pltpu.make_async_copy(
                    e_hbm.at[pl.ds(0, CH2)], e_b.at[b], seme[b]).wait()

            def waitsc(b):
                pltpu.make_async_copy(scaled.at[b],
                                      p_sp.at[dstm.at[0]], sems[b]).wait()

            srcload(0, 0, semr[0])
            srcload(1, 1, semr[1])
            srcwait(0, semr[0])
            mkoff(0)
            gath2(0, 0)

            def chunk2(g, _):
                for b in (0, 1):
                    k = g * 2 + b

                    @pl.when(k >= 1)
                    def _():
                        waitsc(1 - b)

                    @pl.when(k + 1 < nch)
                    def _():
                        srcwait(1 - b, semr[1 - b])
                        mkoff(1 - b)
                        gath2(k + 1, 1 - b)

                    wait2(b)
                    ebuf = e_b.at[b]

                    def row(i2, _):
                        for u in range(2):
                            i = i2 * 2 + u
                            wv = _lane_bcast(ebuf[i], h)
                            for j in range(4):
                                x = rows16[b, i, pl.ds(j * 32, 32)]
                                lo, hi = plsc.unpack(
                                    x, format=plsc.PackFormat.INTERLEAVED)
                                scaled[b, i, pl.ds(j * 32, 16)] = lo * wv
                                scaled[b, i, pl.ds(j * 32 + 16, 16)] = (
                                    hi * wv)
                        return 0

                    lax.fori_loop(0, CH2 // 2, row, 0)
                    pltpu.async_copy(scaled.at[b], p_sp.at[dstm.at[k]],
                                     sems[b], add=True)

                    @pl.when(k + 2 < nch)
                    def _():
                        srcload(k + 2, b, semr[b])
                return 0

            lax.fori_loop(0, nch // 2, chunk2, 0)
            waitsc((nch - 1) % 2)
            plsc.subcore_barrier()
            pltpu.sync_copy(p_sp.at[pl.ds(r0, rpt)],
                            p_hbm.at[cid, h, pl.ds(r0, rpt)])
            return 0

        lax.fori_loop(0, H, head, 0)

    return pl.kernel(
        body,
        out_type=jax.ShapeDtypeStruct((NC, 8, ndp, 128), jnp.float32),
        mesh=mesh,
        scratch_types=[
            pltpu.VMEM((2, 1, CH2), jnp.int32),
            pltpu.VMEM((nch, CH2), jnp.int32),
            pltpu.VMEM((2, CH2, 16), jnp.float32),
            pltpu.VMEM((2, CH2, 128), jnp.bfloat16),
            pltpu.VMEM((2, CH2, 128), jnp.float32),
            pltpu.VMEM_SHARED((ndp, 128), jnp.float32),
        ] + [pltpu.SemaphoreType.DMA] * 8,
        compiler_params=pltpu.CompilerParams(use_tc_tiling_on_sc=False,
                                             needs_layout_passes=False),
    )


def _msg_kernel_allheads(ep, nsrc):
    epw = ep // NW
    CH3 = 16
    nch = epw // CH3
    ndp = 1024
    mesh = plsc.VectorSubcoreMesh(core_axis_name="c", subcore_axis_name="s")

    def body(src_hbm, dst_hbm, e_hbm, hsf_hbm, zerosw_hbm, p_hbm,
             srcr, dstm, e_b, rows16, scaled, p_sp,
             semg0, semg1, seme0, seme1, sems0, sems1, semr0, semr1):
        cid = lax.axis_index("c")
        sid = lax.axis_index("s")
        wid = sid * NC + cid
        rpt = ndp // NS
        r0 = sid * rpt
        semg = (semg0, semg1)
        seme = (seme0, seme1)
        sems = (sems0, sems1)
        semr = (semr0, semr1)

        pltpu.sync_copy(dst_hbm.at[pl.ds(wid * nch, nch)], dstm)
        pltpu.sync_copy(zerosw_hbm.at[pl.ds(r0, rpt)],
                        p_sp.at[pl.ds(r0, rpt)])
        plsc.subcore_barrier()

        def srcload(k, b, sem):
            pltpu.async_copy(src_hbm.at[pl.ds(wid * nch + k, 1)],
                             srcr.at[b], sem)

        def srcwait(b, sem):
            pltpu.make_async_copy(src_hbm.at[pl.ds(0, 1)],
                                  srcr.at[b], sem).wait()

        def gath2(k, b):
            pltpu.async_copy(hsf_hbm.at[srcr.at[b, 0]], rows16.at[b],
                             semg[b])
            pltpu.async_copy(e_hbm.at[pl.ds((wid * nch + k) * CH3, CH3)],
                             e_b.at[b], seme[b])

        def wait2(b):
            pltpu.make_async_copy(hsf_hbm.at[srcr.at[0, 0]],
                                  rows16.at[b], semg[b]).wait()
            pltpu.make_async_copy(
                e_hbm.at[pl.ds(0, CH3)], e_b.at[b], seme[b]).wait()

        def waitsc(b):
            pltpu.make_async_copy(scaled.at[b],
                                  p_sp.at[dstm.at[0]], sems[b]).wait()

        srcload(0, 0, semr[0])
        srcload(1, 1, semr[1])
        srcwait(0, semr[0])
        gath2(0, 0)

        def chunk2(g, _):
            for b in (0, 1):
                k = g * 2 + b

                @pl.when(k >= 1)
                def _():
                    waitsc(1 - b)

                @pl.when(k + 1 < nch)
                def _():
                    srcwait(1 - b, semr[1 - b])
                    gath2(k + 1, 1 - b)

                wait2(b)
                ebuf = e_b.at[b]

                def row(i, _):
                    for hh in range(8):
                        wv = _lane_bcast(ebuf[i], hh)
                        for j in range(4):
                            c0 = hh * 128 + j * 32
                            x = rows16[b, i, pl.ds(c0, 32)]
                            lo, hi = plsc.unpack(
                                x, format=plsc.PackFormat.INTERLEAVED)
                            scaled[b, i, pl.ds(c0, 16)] = lo * wv
                            scaled[b, i, pl.ds(c0 + 16, 16)] = hi * wv
                    return 0

                lax.fori_loop(0, CH3, row, 0)
                pltpu.async_copy(scaled.at[b], p_sp.at[dstm.at[k]],
                                 sems[b], add=True)

                @pl.when(k + 2 < nch)
                def _():
                    srcload(k + 2, b, semr[b])
            return 0

        lax.fori_loop(0, nch // 2, chunk2, 0)
        waitsc((nch - 1) % 2)
        plsc.subcore_barrier()
        pltpu.sync_copy(p_sp.at[pl.ds(r0, rpt)],
                        p_hbm.at[cid, pl.ds(r0, rpt)])

    return pl.kernel(
        body,
        out_type=jax.ShapeDtypeStruct((NC, 1024, 1024), jnp.float32),
        mesh=mesh,
        scratch_types=[
            pltpu.VMEM((2, 1, 16), jnp.int32),
            pltpu.VMEM((epw // 16, 16), jnp.int32),
            pltpu.VMEM((2, 16, 16), jnp.float32),
            pltpu.VMEM((2, 16, 1024), jnp.bfloat16),
            pltpu.VMEM((2, 16, 1024), jnp.float32),
            pltpu.VMEM_SHARED((1024, 1024), jnp.float32),
        ] + [pltpu.SemaphoreType.DMA] * 8,
        compiler_params=pltpu.CompilerParams(use_tc_tiling_on_sc=False,
                                             needs_layout_passes=False),
    )


def _shuffle16(x):
    n, c = x.shape
    x = x.reshape(n, c // 32, 2, 16).transpose(0, 1, 3, 2)
    return x.reshape(n, c).astype(jnp.bfloat16)


def _collapse(w, a):
    return jnp.einsum('dhc,hc->dh', w.reshape(D, H, C), a)


def _a2t(a_src, a_dst):
    z = jnp.zeros((H, C, 16), jnp.float32)
    z = z.at[jnp.arange(H), :, jnp.arange(H)].set(a_src)
    z = z.at[jnp.arange(H), :, 8 + jnp.arange(H)].set(a_dst)
    return z.reshape(HID, 16)


def _pad_edges(edge, ep, dummy_dst):
    e = edge.shape[1]
    src = jnp.pad(edge[0].astype(jnp.int32), (0, ep - e))
    dst = jnp.pad(edge[1].astype(jnp.int32), (0, ep - e),
                  constant_values=dummy_dst)
    return src, dst


def _headmajor16(hlin):
    n = hlin.shape[0]
    x = hlin.reshape(n, H, C).transpose(1, 0, 2).reshape(H * n, C)
    x = x.reshape(H * n, 4, 2, 16).transpose(0, 1, 3, 2)
    return x.reshape(H * n, C).astype(jnp.bfloat16)


def kernel(Hs, Hw, HS, w2s, s2s, S2s, gw_Ws, gw_Wd, gw_as, gw_ad, gw_b,
           gs_W, gs_as, gs_ad, gs_b, gS_Ws, gS_Wd, gS_as, gS_ad, gS_b,
           f1_W, f1_b, f2_W, f2_b, ffn_W1, ffn_b1, ffn_W2, ffn_b2):
    Ns = Hs.shape[0]
    NSec = HS.shape[0]
    zeros = jnp.zeros((NDP, 128), jnp.float32)
    zeros16 = jnp.zeros((NDP, 16), jnp.float32)

    hlw, alw = _proj(Hw[:Ns], gw_Ws, _a2t(gw_as, jnp.zeros_like(gw_as)), 1000)
    hls, als16 = _proj(Hs, gs_W, _a2t(gs_as, gs_ad), 1000)
    hlS, alS = _proj(HS, gS_Ws, _a2t(gS_as, jnp.zeros_like(gS_as)), 1000)

    wd16 = jnp.concatenate(
        [_collapse(gw_Wd, gw_ad), _collapse(gS_Wd, gS_ad)], axis=1)
    ald16 = _mm16(Hs, wd16, 1000)

    def pad_rows(x):
        return jnp.pad(x, ((0, NDP - x.shape[0]), (0, 0)))

    zpad = jnp.zeros((Ns, 8), jnp.float32)
    ald_w = pad_rows(jnp.concatenate([ald16[:, 0:8], zpad], axis=1))
    ald_s = pad_rows(jnp.concatenate([als16[:, 8:16], zpad], axis=1))
    ald_S = pad_rows(jnp.concatenate([ald16[:, 8:16], zpad], axis=1))

    def run_rel(edge, als_rows, ald_rows, hlin, nsrc, ndp):
        ep = _round_up(edge.shape[1], NW * CH * 2)
        src, dst = _pad_edges(edge, ep, ndp - 1)
        e, den = _logit_kernel(ep, ndp)(
            src.reshape(ep // CH, CH), dst.reshape(ep // CH, CH),
            als_rows, ald_rows, zeros16)
        p = _msg_kernel(ep, nsrc, ndp)(
            src.reshape(ep // CH2, CH2), dst.reshape(ep // CH2, CH2),
            e, _headmajor16(hlin), zeros)
        return den, p

    den_w, p_w = run_rel(w2s, alw, ald_w, hlw, Ns, NDP)
    den_s, p_s = run_rel(s2s, als16, ald_s, hls, Ns, NDP)
    epS = _round_up(S2s.shape[1], NW * CH * 2)
    srcS, dstS = _pad_edges(S2s, epS, 1023)
    eS, den_S = _logit_kernel(epS, 1024)(
        srcS.reshape(epS // CH, CH), dstS.reshape(epS // CH, CH),
        alS, ald_S, zeros16)
    zerosw = jnp.zeros((1024, 1024), jnp.float32)
    p_S = _msg_kernel_allheads(epS, NSec)(
        srcS.reshape(epS // 16, 16), dstS.reshape(epS // 16, 16),
        eS, _shuffle16(hlS), zerosw)
    p_S = p_S.reshape(NC, 1024, 8, 128).transpose(0, 2, 1, 3)
    p_S = jnp.pad(p_S[:, :, :NSec], ((0, 0), (0, 0), (0, Ns - NSec), (0, 0)))
    den_S = jnp.pad(den_S[:, :NSec], ((0, 0), (0, Ns - NSec), (0, 0)))

    return _post(Hs,
                 p_w[:, :, :Ns], den_w[:, :Ns], p_s[:, :, :Ns],
                 den_s[:, :Ns], p_S, den_S,
                 gw_b, gs_b, gS_b,
                 f1_W[:HID], f1_W[HID:], f1_b,
                 f2_W[:HID], f2_W[HID:], f2_b,
                 ffn_W1, ffn_b1, ffn_W2, ffn_b2, 400)
